# Initial kernel scaffold; baseline (speedup 1.0000x reference)
import jax
import jax.numpy as jnp
from jax.experimental import pallas as pl


def kernel(objectness_0, centerness_0, location_0, scale_0, objectness_1, centerness_1, location_1, scale_1, objectness_2, centerness_2, location_2, scale_2, objectness_3, centerness_3, location_3, scale_3):
    raise NotImplementedError("write your pallas kernel here")


# speedup vs baseline: 1.0594x; 1.0594x over previous
import jax
import jax.numpy as jnp
from jax import lax
from jax.experimental import pallas as pl
from jax.experimental.pallas import tpu as pltpu

STRIDES = [8, 16, 32, 64]
LEVEL_HW = [(128, 128), (64, 64), (32, 32), (16, 16)]
SIZES = [h * w for h, w in LEVEL_HW]
OFFS = [0, 16384, 20480, 21504]
LOG2W = {128: 7, 64: 6, 32: 5, 16: 4}
N = 21760
B = 16
K = 2000


def _prop_body(o_ref, c_ref, lx_ref, ly_ref, sx_ref, sy_ref,
               s_ref, cx_ref, cy_ref, w_ref, h_ref):
    for l in range(4):
        off = OFFS[l]
        H, W = LEVEL_HW[l]
        hw = H * W
        st = float(STRIDES[l])
        i = lax.broadcasted_iota(jnp.int32, (1, hw), 1)
        x = (i & (W - 1)).astype(jnp.float32)
        y = (i >> LOG2W[W]).astype(jnp.float32)
        xs = (x + 0.5) * st
        ys = (y + 0.5) * st
        sl = (slice(None), pl.ds(off, hw))
        cx_ref[sl] = xs + lx_ref[sl] * st
        cy_ref[sl] = ys + ly_ref[sl] * st
        w_ref[sl] = sx_ref[sl] * st
        h_ref[sl] = sy_ref[sl] * st
        s_ref[sl] = jnp.sqrt(o_ref[sl] * c_ref[sl])


def kernel(objectness_0, centerness_0, location_0, scale_0,
           objectness_1, centerness_1, location_1, scale_1,
           objectness_2, centerness_2, location_2, scale_2,
           objectness_3, centerness_3, location_3, scale_3):
    objs = [objectness_0, objectness_1, objectness_2, objectness_3]
    ctrs = [centerness_0, centerness_1, centerness_2, centerness_3]
    locs = [location_0, location_1, location_2, location_3]
    scls = [scale_0, scale_1, scale_2, scale_3]

    o_all = jnp.concatenate([o.reshape(B, -1) for o in objs], axis=1)
    c_all = jnp.concatenate([c.reshape(B, -1) for c in ctrs], axis=1)
    lx_all = jnp.concatenate([l[:, 0].reshape(B, -1) for l in locs], axis=1)
    ly_all = jnp.concatenate([l[:, 1].reshape(B, -1) for l in locs], axis=1)
    sx_all = jnp.concatenate([s[:, 0].reshape(B, -1) for s in scls], axis=1)
    sy_all = jnp.concatenate([s[:, 1].reshape(B, -1) for s in scls], axis=1)

    out_sh = [jax.ShapeDtypeStruct((B, N), jnp.float32)] * 5
    s_all, cx_all, cy_all, w_all, h_all = pl.pallas_call(
        _prop_body,
        out_shape=out_sh,
    )(o_all, c_all, lx_all, ly_all, sx_all, sy_all)

    _, idx = lax.top_k(s_all, K)
    take = lambda a: jnp.take_along_axis(a, idx, axis=1)
    return jnp.stack([take(cx_all), take(cy_all), take(w_all), take(h_all),
                      take(s_all)], axis=-1)


# speedup vs baseline: 2.7157x; 2.5635x over previous
import functools

import jax
import jax.numpy as jnp
from jax import lax
from jax.experimental import pallas as pl
from jax.experimental.pallas import tpu as pltpu
from jax.experimental.pallas import tpu_sc as plsc

STRIDES = [8, 16, 32, 64]
LEVEL_HW = [(128, 128), (64, 64), (32, 32), (16, 16)]
OFFS = [0, 16384, 20480, 21504]
LOG2W = {128: 7, 64: 6, 32: 5, 16: 4}
N = 21760
NV = N // 16
B = 16
K = 2000
KV = K // 16
CAP = 4096
H1 = 4096


def _score_body(o_ref, c_ref, s_ref):
    s_ref[...] = jnp.sqrt(o_ref[...] * c_ref[...])


def _iota16():
    return lax.broadcasted_iota(jnp.int32, (16,), 0)


def _splat(x):
    return jnp.full((16,), x, dtype=jnp.int32)


def _sc_body(s_hbm, lx_hbm, ly_hbm, sx_hbm, sy_hbm,
             ocx, ocy, ow, oh, osc,
             sbuf, hist1, cb_a, ci_a, cb_b, ci_b, offs, idxg,
             g0, g1, g2, g3, gs, sem):
    cid = lax.axis_index("c")
    sid = lax.axis_index("s")
    row = sid * 2 + cid

    @pl.when(sid < 8)
    def _():
        iota = _iota16()
        zero16 = jnp.zeros((16,), jnp.int32)
        ones16 = jnp.ones((16,), jnp.int32)

        pltpu.sync_copy(s_hbm.at[row], sbuf)

        def z_body(i, c):
            hist1[pl.ds(i * 16, 16)] = zero16
            return c
        lax.fori_loop(0, H1 // 16, z_body, 0)

        def h_body(i, c):
            bits = plsc.bitcast(sbuf[pl.ds(i * 16, 16)], jnp.int32)
            d1 = bits >> 18
            cnt, last = plsc.scan_count(d1)
            plsc.addupdate_scatter(hist1, [d1], cnt, mask=last)
            return c
        lax.fori_loop(0, NV, h_body, 0)

        def t_cond(st):
            j, cum, t1 = st
            return jnp.logical_and(cum < K, j >= 0)

        def t_body(st):
            j, cum, t1 = st
            v = hist1[pl.ds(j * 16, 16)]
            rv = lax.rev(v, (0,))
            csum = plsc.cumsum(rv) + cum
            buckets = j * 16 + 15 - iota
            t1c = jnp.max(jnp.where(csum >= K, buckets, -1))
            t1 = jnp.maximum(t1, t1c)
            cum = cum + jnp.max(csum)
            return j - 1, cum, t1

        _, _, t1 = lax.while_loop(
            t_cond, t_body, (H1 // 16 - 1, jnp.int32(0), jnp.int32(-1)))

        def c_body(i, cnt_splat):
            bits = plsc.bitcast(sbuf[pl.ds(i * 16, 16)], jnp.int32)
            d1 = bits >> 18
            m = d1 >= t1
            pos = cnt_splat + plsc.cumsum(ones16, mask=m) - 1
            safe = jnp.logical_and(m, pos < CAP)
            plsc.store_scatter(cb_a, [pos], bits, mask=safe)
            plsc.store_scatter(ci_a, [pos], i * 16 + iota, mask=safe)
            nm = plsc.all_reduce_population_count(m)
            return cnt_splat + nm
        cnt_splat = lax.fori_loop(0, NV, c_body, zero16)
        csel = jnp.minimum(jnp.max(cnt_splat), CAP)

        base = csel & ~15
        pad_m = jnp.logical_and(base + iota >= csel, base + iota < CAP)
        plsc.store_scatter(cb_a, [base + iota], zero16, mask=pad_m)
        plsc.store_scatter(ci_a, [base + iota], _splat(N), mask=pad_m)
        nvc = (csel + 15) >> 4

        bufs = [(cb_a, ci_a, cb_b, ci_b), (cb_b, ci_b, cb_a, ci_a),
                (cb_a, ci_a, cb_b, ci_b), (cb_b, ci_b, cb_a, ci_a)]
        for p in range(4):
            src_b, src_i, dst_b, dst_i = bufs[p]
            shift = 8 * p

            def rz_body(j, c):
                offs[pl.ds(j * 16, 16)] = zero16
                return c
            lax.fori_loop(0, 16, rz_body, 0)

            def rh_body(i, c, src_b=src_b, shift=shift):
                bits = src_b[pl.ds(i * 16, 16)]
                d = (jnp.invert(bits) >> shift) & 255
                cnt, last = plsc.scan_count(d)
                plsc.addupdate_scatter(offs, [d], cnt, mask=last)
                return c
            lax.fori_loop(0, nvc, rh_body, 0)

            def rs_body(j, carry):
                v = offs[pl.ds(j * 16, 16)]
                csum = plsc.cumsum(v)
                offs[pl.ds(j * 16, 16)] = csum - v + carry
                return carry + jnp.full((16,), jnp.max(csum), jnp.int32)
            lax.fori_loop(0, 16, rs_body, zero16)

            def rp_body(i, c, src_b=src_b, src_i=src_i, dst_b=dst_b,
                        dst_i=dst_i, shift=shift):
                bits = src_b[pl.ds(i * 16, 16)]
                idxv = src_i[pl.ds(i * 16, 16)]
                d = (jnp.invert(bits) >> shift) & 255
                cnt, last = plsc.scan_count(d)
                dest = plsc.load_gather(offs, [d]) + cnt - 1
                plsc.store_scatter(dst_b, [dest], bits)
                plsc.store_scatter(dst_i, [dest], idxv)
                plsc.addupdate_scatter(offs, [d], cnt, mask=last)
                return c
            lax.fori_loop(0, nvc, rp_body, 0)

        def g_body(i, c):
            idxg[pl.ds(i * 16, 16)] = ci_a[pl.ds(i * 16, 16)] + row * N
            return c
        lax.fori_loop(0, KV, g_body, 0)

        cp0 = pltpu.async_copy(lx_hbm.at[idxg], g0, sem)
        cp1 = pltpu.async_copy(ly_hbm.at[idxg], g1, sem)
        cp2 = pltpu.async_copy(sx_hbm.at[idxg], g2, sem)
        cp3 = pltpu.async_copy(sy_hbm.at[idxg], g3, sem)
        cp0.wait(); cp1.wait(); cp2.wait(); cp3.wait()

        def o_body(i, c):
            sl = pl.ds(i * 16, 16)
            idxv = ci_a[sl]
            bits = cb_a[sl]
            lvl = ((idxv >= OFFS[1]).astype(jnp.int32)
                   + (idxv >= OFFS[2]).astype(jnp.int32)
                   + (idxv >= OFFS[3]).astype(jnp.int32))
            off = jnp.where(lvl == 0, OFFS[0],
                  jnp.where(lvl == 1, OFFS[1],
                  jnp.where(lvl == 2, OFFS[2], OFFS[3])))
            log2w = 7 - lvl
            r = idxv - off
            x = (r & ((1 << log2w) - 1)).astype(jnp.float32)
            y = (r >> log2w).astype(jnp.float32)
            stf = (8 << lvl).astype(jnp.float32)
            g0[sl] = (x + 0.5) * stf + g0[sl] * stf
            g1[sl] = (y + 0.5) * stf + g1[sl] * stf
            g2[sl] = g2[sl] * stf
            g3[sl] = g3[sl] * stf
            gs[sl] = plsc.bitcast(bits, jnp.float32)
            return c
        lax.fori_loop(0, KV, o_body, 0)

        pltpu.sync_copy(g0, ocx.at[row])
        pltpu.sync_copy(g1, ocy.at[row])
        pltpu.sync_copy(g2, ow.at[row])
        pltpu.sync_copy(g3, oh.at[row])
        pltpu.sync_copy(gs, osc.at[row])


def kernel(objectness_0, centerness_0, location_0, scale_0,
           objectness_1, centerness_1, location_1, scale_1,
           objectness_2, centerness_2, location_2, scale_2,
           objectness_3, centerness_3, location_3, scale_3):
    objs = [objectness_0, objectness_1, objectness_2, objectness_3]
    ctrs = [centerness_0, centerness_1, centerness_2, centerness_3]
    locs = [location_0, location_1, location_2, location_3]
    scls = [scale_0, scale_1, scale_2, scale_3]

    o_all = jnp.concatenate([o.reshape(B, -1) for o in objs], axis=1)
    c_all = jnp.concatenate([c.reshape(B, -1) for c in ctrs], axis=1)
    lx = jnp.concatenate([l[:, 0].reshape(B, -1) for l in locs], axis=1).reshape(-1)
    ly = jnp.concatenate([l[:, 1].reshape(B, -1) for l in locs], axis=1).reshape(-1)
    sx = jnp.concatenate([s[:, 0].reshape(B, -1) for s in scls], axis=1).reshape(-1)
    sy = jnp.concatenate([s[:, 1].reshape(B, -1) for s in scls], axis=1).reshape(-1)

    s_all = pl.pallas_call(
        _score_body,
        out_shape=jax.ShapeDtypeStruct((B, N), jnp.float32),
    )(o_all, c_all)

    mesh = plsc.VectorSubcoreMesh(core_axis_name="c", subcore_axis_name="s")
    out_t = [jax.ShapeDtypeStruct((B, K), jnp.float32)] * 5
    sc_topk = pl.kernel(
        _sc_body,
        out_type=out_t,
        mesh=mesh,
        compiler_params=pltpu.CompilerParams(needs_layout_passes=False),
        scratch_types=[
            pltpu.VMEM((N,), jnp.float32),
            pltpu.VMEM((H1,), jnp.int32),
            pltpu.VMEM((CAP,), jnp.int32),
            pltpu.VMEM((CAP,), jnp.int32),
            pltpu.VMEM((CAP,), jnp.int32),
            pltpu.VMEM((CAP,), jnp.int32),
            pltpu.VMEM((256,), jnp.int32),
            pltpu.VMEM((K,), jnp.int32),
            pltpu.VMEM((K,), jnp.float32),
            pltpu.VMEM((K,), jnp.float32),
            pltpu.VMEM((K,), jnp.float32),
            pltpu.VMEM((K,), jnp.float32),
            pltpu.VMEM((K,), jnp.float32),
            pltpu.SemaphoreType.DMA,
        ],
    )
    cx, cy, w, h, s = sc_topk(s_all, lx, ly, sx, sy)
    return jnp.stack([cx, cy, w, h, s], axis=-1)


# speedup vs baseline: 3.1670x; 1.1662x over previous
import functools

import jax
import jax.numpy as jnp
from jax import lax
from jax.experimental import pallas as pl
from jax.experimental.pallas import tpu as pltpu
from jax.experimental.pallas import tpu_sc as plsc

STRIDES = [8, 16, 32, 64]
LEVEL_HW = [(128, 128), (64, 64), (32, 32), (16, 16)]
OFFS = [0, 16384, 20480, 21504]
LOG2W = {128: 7, 64: 6, 32: 5, 16: 4}
N = 21760
NV = N // 16
B = 16
K = 2000
KV = K // 16
CAP = 4096
H1 = 4096


def _score_body(*refs):
    o = refs[0:4]
    c = refs[4:8]
    locr = refs[8:12]
    sclr = refs[12:16]
    s_ref, lx_ref, ly_ref, sx_ref, sy_ref = refs[16:21]
    for l in range(4):
        off = OFFS[l]
        hw = LEVEL_HW[l][0] * LEVEL_HW[l][1]
        sl = (slice(None), pl.ds(off, hw))
        s_ref[sl] = jnp.sqrt(o[l][...].reshape(B, hw) * c[l][...].reshape(B, hw))
        lx_ref[sl] = locr[l][:, 0].reshape(B, hw)
        ly_ref[sl] = locr[l][:, 1].reshape(B, hw)
        sx_ref[sl] = sclr[l][:, 0].reshape(B, hw)
        sy_ref[sl] = sclr[l][:, 1].reshape(B, hw)


def _iota16():
    return lax.broadcasted_iota(jnp.int32, (16,), 0)


def _splat(x):
    return jnp.full((16,), x, dtype=jnp.int32)


def _sc_body(s_hbm, lx_hbm, ly_hbm, sx_hbm, sy_hbm,
             ocx, ocy, ow, oh, osc,
             sbuf, hist1, cb_a, ci_a, cb_b, ci_b, offs, idxg,
             g0, g1, g2, g3, gs, sem):
    cid = lax.axis_index("c")
    sid = lax.axis_index("s")
    row = sid * 2 + cid

    @pl.when(sid < 8)
    def _():
        iota = _iota16()
        zero16 = jnp.zeros((16,), jnp.int32)
        ones16 = jnp.ones((16,), jnp.int32)

        pltpu.sync_copy(s_hbm.at[row], sbuf)

        def z_body(i, c):
            hist1[pl.ds(i * 16, 16)] = zero16
            return c
        lax.fori_loop(0, H1 // 16, z_body, 0)

        def h_body(i, c):
            bits = plsc.bitcast(sbuf[pl.ds(i * 16, 16)], jnp.int32)
            d1 = bits >> 18
            cnt, last = plsc.scan_count(d1)
            plsc.addupdate_scatter(hist1, [d1], cnt, mask=last)
            return c
        lax.fori_loop(0, NV, h_body, 0)

        def t_cond(st):
            j, cum, t1 = st
            return jnp.logical_and(cum < K, j >= 0)

        def t_body(st):
            j, cum, t1 = st
            v = hist1[pl.ds(j * 16, 16)]
            rv = lax.rev(v, (0,))
            csum = plsc.cumsum(rv) + cum
            buckets = j * 16 + 15 - iota
            t1c = jnp.max(jnp.where(csum >= K, buckets, -1))
            t1 = jnp.maximum(t1, t1c)
            cum = cum + jnp.max(csum)
            return j - 1, cum, t1

        _, _, t1 = lax.while_loop(
            t_cond, t_body, (H1 // 16 - 1, jnp.int32(0), jnp.int32(-1)))

        def c_body(i, cnt_splat):
            bits = plsc.bitcast(sbuf[pl.ds(i * 16, 16)], jnp.int32)
            d1 = bits >> 18
            m = d1 >= t1
            pos = cnt_splat + plsc.cumsum(ones16, mask=m) - 1
            safe = jnp.logical_and(m, pos < CAP)
            plsc.store_scatter(cb_a, [pos], bits, mask=safe)
            plsc.store_scatter(ci_a, [pos], i * 16 + iota, mask=safe)
            nm = plsc.all_reduce_population_count(m)
            return cnt_splat + nm
        cnt_splat = lax.fori_loop(0, NV, c_body, zero16)
        csel = jnp.minimum(jnp.max(cnt_splat), CAP)

        base = csel & ~15
        pad_m = jnp.logical_and(base + iota >= csel, base + iota < CAP)
        plsc.store_scatter(cb_a, [base + iota], zero16, mask=pad_m)
        plsc.store_scatter(ci_a, [base + iota], _splat(N), mask=pad_m)
        nvc = (csel + 15) >> 4

        bufs = [(cb_a, ci_a, cb_b, ci_b), (cb_b, ci_b, cb_a, ci_a),
                (cb_a, ci_a, cb_b, ci_b), (cb_b, ci_b, cb_a, ci_a)]
        for p in range(4):
            src_b, src_i, dst_b, dst_i = bufs[p]
            shift = 8 * p

            def rz_body(j, c):
                offs[pl.ds(j * 16, 16)] = zero16
                return c
            lax.fori_loop(0, 16, rz_body, 0)

            def rh_body(i, c, src_b=src_b, shift=shift):
                bits = src_b[pl.ds(i * 16, 16)]
                d = (jnp.invert(bits) >> shift) & 255
                cnt, last = plsc.scan_count(d)
                plsc.addupdate_scatter(offs, [d], cnt, mask=last)
                return c
            lax.fori_loop(0, nvc, rh_body, 0)

            def rs_body(j, carry):
                v = offs[pl.ds(j * 16, 16)]
                csum = plsc.cumsum(v)
                offs[pl.ds(j * 16, 16)] = csum - v + carry
                return carry + jnp.full((16,), jnp.max(csum), jnp.int32)
            lax.fori_loop(0, 16, rs_body, zero16)

            def rp_body(i, c, src_b=src_b, src_i=src_i, dst_b=dst_b,
                        dst_i=dst_i, shift=shift):
                bits = src_b[pl.ds(i * 16, 16)]
                idxv = src_i[pl.ds(i * 16, 16)]
                d = (jnp.invert(bits) >> shift) & 255
                cnt, last = plsc.scan_count(d)
                dest = plsc.load_gather(offs, [d]) + cnt - 1
                plsc.store_scatter(dst_b, [dest], bits)
                plsc.store_scatter(dst_i, [dest], idxv)
                plsc.addupdate_scatter(offs, [d], cnt, mask=last)
                return c
            lax.fori_loop(0, nvc, rp_body, 0)

        def g_body(i, c):
            idxg[pl.ds(i * 16, 16)] = ci_a[pl.ds(i * 16, 16)] + row * N
            return c
        lax.fori_loop(0, KV, g_body, 0)

        cp0 = pltpu.async_copy(lx_hbm.at[idxg], g0, sem)
        cp1 = pltpu.async_copy(ly_hbm.at[idxg], g1, sem)
        cp2 = pltpu.async_copy(sx_hbm.at[idxg], g2, sem)
        cp3 = pltpu.async_copy(sy_hbm.at[idxg], g3, sem)
        cp0.wait(); cp1.wait(); cp2.wait(); cp3.wait()

        def o_body(i, c):
            sl = pl.ds(i * 16, 16)
            idxv = ci_a[sl]
            bits = cb_a[sl]
            lvl = ((idxv >= OFFS[1]).astype(jnp.int32)
                   + (idxv >= OFFS[2]).astype(jnp.int32)
                   + (idxv >= OFFS[3]).astype(jnp.int32))
            off = jnp.where(lvl == 0, OFFS[0],
                  jnp.where(lvl == 1, OFFS[1],
                  jnp.where(lvl == 2, OFFS[2], OFFS[3])))
            log2w = 7 - lvl
            r = idxv - off
            x = (r & ((1 << log2w) - 1)).astype(jnp.float32)
            y = (r >> log2w).astype(jnp.float32)
            stf = (8 << lvl).astype(jnp.float32)
            g0[sl] = (x + 0.5) * stf + g0[sl] * stf
            g1[sl] = (y + 0.5) * stf + g1[sl] * stf
            g2[sl] = g2[sl] * stf
            g3[sl] = g3[sl] * stf
            gs[sl] = plsc.bitcast(bits, jnp.float32)
            return c
        lax.fori_loop(0, KV, o_body, 0)

        pltpu.sync_copy(g0, ocx.at[row])
        pltpu.sync_copy(g1, ocy.at[row])
        pltpu.sync_copy(g2, ow.at[row])
        pltpu.sync_copy(g3, oh.at[row])
        pltpu.sync_copy(gs, osc.at[row])


def kernel(objectness_0, centerness_0, location_0, scale_0,
           objectness_1, centerness_1, location_1, scale_1,
           objectness_2, centerness_2, location_2, scale_2,
           objectness_3, centerness_3, location_3, scale_3):
    objs = [objectness_0, objectness_1, objectness_2, objectness_3]
    ctrs = [centerness_0, centerness_1, centerness_2, centerness_3]
    locs = [location_0, location_1, location_2, location_3]
    scls = [scale_0, scale_1, scale_2, scale_3]

    s_all, lx2, ly2, sx2, sy2 = pl.pallas_call(
        _score_body,
        out_shape=[jax.ShapeDtypeStruct((B, N), jnp.float32)] * 5,
    )(*objs, *ctrs, *locs, *scls)
    lx, ly, sx, sy = (a.reshape(-1) for a in (lx2, ly2, sx2, sy2))

    mesh = plsc.VectorSubcoreMesh(core_axis_name="c", subcore_axis_name="s")
    out_t = [jax.ShapeDtypeStruct((B, K), jnp.float32)] * 5
    sc_topk = pl.kernel(
        _sc_body,
        out_type=out_t,
        mesh=mesh,
        compiler_params=pltpu.CompilerParams(needs_layout_passes=False),
        scratch_types=[
            pltpu.VMEM((N,), jnp.float32),
            pltpu.VMEM((H1,), jnp.int32),
            pltpu.VMEM((CAP,), jnp.int32),
            pltpu.VMEM((CAP,), jnp.int32),
            pltpu.VMEM((CAP,), jnp.int32),
            pltpu.VMEM((CAP,), jnp.int32),
            pltpu.VMEM((256,), jnp.int32),
            pltpu.VMEM((K,), jnp.int32),
            pltpu.VMEM((K,), jnp.float32),
            pltpu.VMEM((K,), jnp.float32),
            pltpu.VMEM((K,), jnp.float32),
            pltpu.VMEM((K,), jnp.float32),
            pltpu.VMEM((K,), jnp.float32),
            pltpu.SemaphoreType.DMA,
        ],
    )
    cx, cy, w, h, s = sc_topk(s_all, lx, ly, sx, sy)
    return jnp.stack([cx, cy, w, h, s], axis=-1)


# speedup vs baseline: 3.7419x; 1.1816x over previous
import functools

import jax
import jax.numpy as jnp
from jax import lax
from jax.experimental import pallas as pl
from jax.experimental.pallas import tpu as pltpu
from jax.experimental.pallas import tpu_sc as plsc

STRIDES = [8, 16, 32, 64]
LEVEL_HW = [(128, 128), (64, 64), (32, 32), (16, 16)]
OFFS = [0, 16384, 20480, 21504]
LOG2W = {128: 7, 64: 6, 32: 5, 16: 4}
N = 21760
N2 = N // 2
NV2 = N2 // 16
B = 16
K = 2000
KV = K // 16
CAPH = 3072
CAPT = 2 * CAPH
H1 = 4096


def _score_body(*refs):
    o = refs[0:4]
    c = refs[4:8]
    locr = refs[8:12]
    sclr = refs[12:16]
    s_ref, lx_ref, ly_ref, sx_ref, sy_ref = refs[16:21]
    for l in range(4):
        off = OFFS[l]
        hw = LEVEL_HW[l][0] * LEVEL_HW[l][1]
        sl = (slice(None), pl.ds(off, hw))
        s_ref[sl] = jnp.sqrt(o[l][...].reshape(B, hw) * c[l][...].reshape(B, hw))
        lx_ref[sl] = locr[l][:, 0].reshape(B, hw)
        ly_ref[sl] = locr[l][:, 1].reshape(B, hw)
        sx_ref[sl] = sclr[l][:, 0].reshape(B, hw)
        sy_ref[sl] = sclr[l][:, 1].reshape(B, hw)


def _iota16():
    return lax.broadcasted_iota(jnp.int32, (16,), 0)


def _splat(x):
    return jnp.full((16,), x, dtype=jnp.int32)


def _sc_body(s_hbm, lx_hbm, ly_hbm, sx_hbm, sy_hbm,
             ocx, ocy, ow, oh, osc,
             sbuf, hist1, hist_p, cntbuf,
             cb_a, ci_a, cb_b, ci_b, offs, idxg,
             g0, g1, g2, g3, gs,
             sh_hist, sh_cb, sh_ci, sh_cnt, sem):
    cid = lax.axis_index("c")
    sid = lax.axis_index("s")
    row = cid * 8 + (sid >> 1)
    half = sid & 1

    iota = _iota16()
    zero16 = jnp.zeros((16,), jnp.int32)
    ones16 = jnp.ones((16,), jnp.int32)

    pltpu.sync_copy(s_hbm.at[row * 2 + half], sbuf)

    def z_body(i, c):
        hist1[pl.ds(i * 16, 16)] = zero16
        return c
    lax.fori_loop(0, H1 // 16, z_body, 0)

    def h_body(i, c):
        bits = plsc.bitcast(sbuf[pl.ds(i * 16, 16)], jnp.int32)
        d1 = bits >> 18
        cnt, last = plsc.scan_count(d1)
        plsc.addupdate_scatter(hist1, [d1], cnt, mask=last)
        return c
    lax.fori_loop(0, NV2, h_body, 0)

    pltpu.sync_copy(hist1, sh_hist.at[sid])
    plsc.subcore_barrier()
    pltpu.sync_copy(sh_hist.at[sid ^ 1], hist_p)

    def t_cond(st):
        j, cum, t1 = st
        return jnp.logical_and(cum < K, j >= 0)

    def t_body(st):
        j, cum, t1 = st
        v = hist1[pl.ds(j * 16, 16)] + hist_p[pl.ds(j * 16, 16)]
        rv = lax.rev(v, (0,))
        csum = plsc.cumsum(rv) + cum
        buckets = j * 16 + 15 - iota
        t1c = jnp.max(jnp.where(csum >= K, buckets, -1))
        t1 = jnp.maximum(t1, t1c)
        cum = cum + jnp.max(csum)
        return j - 1, cum, t1

    _, _, t1 = lax.while_loop(
        t_cond, t_body, (H1 // 16 - 1, jnp.int32(0), jnp.int32(-1)))

    ibase = half * N2

    def c_body(i, cnt_splat):
        bits = plsc.bitcast(sbuf[pl.ds(i * 16, 16)], jnp.int32)
        d1 = bits >> 18
        m = d1 >= t1
        pos = cnt_splat + plsc.cumsum(ones16, mask=m) - 1
        safe = jnp.logical_and(m, pos < CAPH)
        plsc.store_scatter(cb_a, [pos], bits, mask=safe)
        plsc.store_scatter(ci_a, [pos], ibase + i * 16 + iota, mask=safe)
        nm = plsc.all_reduce_population_count(m)
        return cnt_splat + nm
    cnt_splat = lax.fori_loop(0, NV2, c_body, zero16)
    cntbuf[pl.ds(0, 16)] = jnp.minimum(cnt_splat, CAPH)

    @pl.when(half == 1)
    def _():
        pltpu.sync_copy(cb_a.at[pl.ds(0, CAPH)], sh_cb.at[sid])
        pltpu.sync_copy(ci_a.at[pl.ds(0, CAPH)], sh_ci.at[sid])
        pltpu.sync_copy(cntbuf, sh_cnt.at[sid])
    plsc.subcore_barrier()

    @pl.when(half == 0)
    def _():
        n_a = jnp.max(cntbuf[pl.ds(0, 16)])
        pltpu.sync_copy(sh_cb.at[sid ^ 1], cb_a.at[pl.ds(CAPH, CAPH)])
        pltpu.sync_copy(sh_ci.at[sid ^ 1], ci_a.at[pl.ds(CAPH, CAPH)])
        pltpu.sync_copy(sh_cnt.at[sid ^ 1], cntbuf)
        n_b = jnp.max(cntbuf[pl.ds(0, 16)])
        csel = n_a + n_b
        nvc = (csel + 15) >> 4

        def rz_body(j, c):
            offs[pl.ds(j * 16, 16)] = zero16
            return c

        def rs_body(j, carry):
            v = offs[pl.ds(j * 16, 16)]
            csum = plsc.cumsum(v)
            offs[pl.ds(j * 16, 16)] = csum - v + carry
            return carry + jnp.full((16,), jnp.max(csum), jnp.int32)

        lax.fori_loop(0, 16, rz_body, 0)

        def seg_hist(i, c, seg_off, seg_n):
            valid = i * 16 + iota < seg_n
            bits = cb_a[pl.ds(seg_off + i * 16, 16)]
            d = jnp.invert(bits) & 255
            cnt, last = plsc.scan_count(d, mask=valid)
            plsc.addupdate_scatter(offs, [d], cnt, mask=last)
            return c
        lax.fori_loop(0, (n_a + 15) >> 4,
                      functools.partial(seg_hist, seg_off=0, seg_n=n_a), 0)
        lax.fori_loop(0, (n_b + 15) >> 4,
                      functools.partial(seg_hist, seg_off=CAPH, seg_n=n_b), 0)

        lax.fori_loop(0, 16, rs_body, zero16)

        def seg_perm(i, c, seg_off, seg_n):
            valid = i * 16 + iota < seg_n
            bits = cb_a[pl.ds(seg_off + i * 16, 16)]
            idxv = ci_a[pl.ds(seg_off + i * 16, 16)]
            d = jnp.invert(bits) & 255
            cnt, last = plsc.scan_count(d, mask=valid)
            dest = plsc.load_gather(offs, [d]) + cnt - 1
            plsc.store_scatter(cb_b, [dest], bits, mask=valid)
            plsc.store_scatter(ci_b, [dest], idxv, mask=valid)
            plsc.addupdate_scatter(offs, [d], cnt, mask=last)
            return c
        lax.fori_loop(0, (n_a + 15) >> 4,
                      functools.partial(seg_perm, seg_off=0, seg_n=n_a), 0)
        lax.fori_loop(0, (n_b + 15) >> 4,
                      functools.partial(seg_perm, seg_off=CAPH, seg_n=n_b), 0)

        base = csel & ~15
        pad_m = jnp.logical_and(base + iota >= csel, base + iota < CAPT)
        plsc.store_scatter(cb_b, [base + iota], zero16, mask=pad_m)
        plsc.store_scatter(ci_b, [base + iota], _splat(N), mask=pad_m)

        bufs = [(cb_b, ci_b, cb_a, ci_a), (cb_a, ci_a, cb_b, ci_b),
                (cb_b, ci_b, cb_a, ci_a)]
        for p in range(1, 4):
            src_b, src_i, dst_b, dst_i = bufs[p - 1]
            shift = 8 * p

            lax.fori_loop(0, 16, rz_body, 0)

            def rh_body(i, c, src_b=src_b, shift=shift):
                bits = src_b[pl.ds(i * 16, 16)]
                d = (jnp.invert(bits) >> shift) & 255
                cnt, last = plsc.scan_count(d)
                plsc.addupdate_scatter(offs, [d], cnt, mask=last)
                return c
            lax.fori_loop(0, nvc, rh_body, 0)

            lax.fori_loop(0, 16, rs_body, zero16)

            def rp_body(i, c, src_b=src_b, src_i=src_i, dst_b=dst_b,
                        dst_i=dst_i, shift=shift):
                bits = src_b[pl.ds(i * 16, 16)]
                idxv = src_i[pl.ds(i * 16, 16)]
                d = (jnp.invert(bits) >> shift) & 255
                cnt, last = plsc.scan_count(d)
                dest = plsc.load_gather(offs, [d]) + cnt - 1
                plsc.store_scatter(dst_b, [dest], bits)
                plsc.store_scatter(dst_i, [dest], idxv)
                plsc.addupdate_scatter(offs, [d], cnt, mask=last)
                return c
            lax.fori_loop(0, nvc, rp_body, 0)

        def g_body(i, c):
            idxg[pl.ds(i * 16, 16)] = ci_a[pl.ds(i * 16, 16)] + row * N
            return c
        lax.fori_loop(0, KV, g_body, 0)

        cp0 = pltpu.async_copy(lx_hbm.at[idxg], g0, sem)
        cp1 = pltpu.async_copy(ly_hbm.at[idxg], g1, sem)
        cp2 = pltpu.async_copy(sx_hbm.at[idxg], g2, sem)
        cp3 = pltpu.async_copy(sy_hbm.at[idxg], g3, sem)
        cp0.wait(); cp1.wait(); cp2.wait(); cp3.wait()

        def o_body(i, c):
            sl = pl.ds(i * 16, 16)
            idxv = ci_a[sl]
            bits = cb_a[sl]
            lvl = ((idxv >= OFFS[1]).astype(jnp.int32)
                   + (idxv >= OFFS[2]).astype(jnp.int32)
                   + (idxv >= OFFS[3]).astype(jnp.int32))
            off = jnp.where(lvl == 0, OFFS[0],
                  jnp.where(lvl == 1, OFFS[1],
                  jnp.where(lvl == 2, OFFS[2], OFFS[3])))
            log2w = 7 - lvl
            r = idxv - off
            x = (r & ((1 << log2w) - 1)).astype(jnp.float32)
            y = (r >> log2w).astype(jnp.float32)
            stf = (8 << lvl).astype(jnp.float32)
            g0[sl] = (x + 0.5) * stf + g0[sl] * stf
            g1[sl] = (y + 0.5) * stf + g1[sl] * stf
            g2[sl] = g2[sl] * stf
            g3[sl] = g3[sl] * stf
            gs[sl] = plsc.bitcast(bits, jnp.float32)
            return c
        lax.fori_loop(0, KV, o_body, 0)

        pltpu.sync_copy(g0, ocx.at[row])
        pltpu.sync_copy(g1, ocy.at[row])
        pltpu.sync_copy(g2, ow.at[row])
        pltpu.sync_copy(g3, oh.at[row])
        pltpu.sync_copy(gs, osc.at[row])


def kernel(objectness_0, centerness_0, location_0, scale_0,
           objectness_1, centerness_1, location_1, scale_1,
           objectness_2, centerness_2, location_2, scale_2,
           objectness_3, centerness_3, location_3, scale_3):
    objs = [objectness_0, objectness_1, objectness_2, objectness_3]
    ctrs = [centerness_0, centerness_1, centerness_2, centerness_3]
    locs = [location_0, location_1, location_2, location_3]
    scls = [scale_0, scale_1, scale_2, scale_3]

    s_all, lx2, ly2, sx2, sy2 = pl.pallas_call(
        _score_body,
        out_shape=[jax.ShapeDtypeStruct((B, N), jnp.float32)] * 5,
    )(*objs, *ctrs, *locs, *scls)
    lx, ly, sx, sy = (a.reshape(-1) for a in (lx2, ly2, sx2, sy2))

    mesh = plsc.VectorSubcoreMesh(core_axis_name="c", subcore_axis_name="s")
    out_t = [jax.ShapeDtypeStruct((B, K), jnp.float32)] * 5
    sc_topk = pl.kernel(
        _sc_body,
        out_type=out_t,
        mesh=mesh,
        compiler_params=pltpu.CompilerParams(needs_layout_passes=False),
        scratch_types=[
            pltpu.VMEM((N2,), jnp.float32),
            pltpu.VMEM((H1,), jnp.int32),
            pltpu.VMEM((H1,), jnp.int32),
            pltpu.VMEM((128,), jnp.int32),
            pltpu.VMEM((CAPT,), jnp.int32),
            pltpu.VMEM((CAPT,), jnp.int32),
            pltpu.VMEM((CAPT,), jnp.int32),
            pltpu.VMEM((CAPT,), jnp.int32),
            pltpu.VMEM((256,), jnp.int32),
            pltpu.VMEM((K,), jnp.int32),
            pltpu.VMEM((K,), jnp.float32),
            pltpu.VMEM((K,), jnp.float32),
            pltpu.VMEM((K,), jnp.float32),
            pltpu.VMEM((K,), jnp.float32),
            pltpu.VMEM((K,), jnp.float32),
            pltpu.VMEM_SHARED((16, H1), jnp.int32),
            pltpu.VMEM_SHARED((16, CAPH), jnp.int32),
            pltpu.VMEM_SHARED((16, CAPH), jnp.int32),
            pltpu.VMEM_SHARED((16, 128), jnp.int32),
            pltpu.SemaphoreType.DMA,
        ],
    )
    cx, cy, w, h, s = sc_topk(s_all.reshape(B * 2, N2), lx, ly, sx, sy)
    return jnp.stack([cx, cy, w, h, s], axis=-1)


# speedup vs baseline: 4.0381x; 1.0791x over previous
import functools

import jax
import jax.numpy as jnp
from jax import lax
from jax.experimental import pallas as pl
from jax.experimental.pallas import tpu as pltpu
from jax.experimental.pallas import tpu_sc as plsc

STRIDES = [8, 16, 32, 64]
LEVEL_HW = [(128, 128), (64, 64), (32, 32), (16, 16)]
OFFS = [0, 16384, 20480, 21504]
LOG2W = {128: 7, 64: 6, 32: 5, 16: 4}
N = 21760
N2 = N // 2
NV2 = N2 // 16
B = 16
K = 2000
KV = K // 16
CAPH = 3072
CAPT = 2 * CAPH
H1 = 4096


def _score_body(*refs):
    o = refs[0:4]
    c = refs[4:8]
    locr = refs[8:12]
    sclr = refs[12:16]
    s_ref, lx_ref, ly_ref, sx_ref, sy_ref = refs[16:21]
    for l in range(4):
        off = OFFS[l]
        hw = LEVEL_HW[l][0] * LEVEL_HW[l][1]
        sl = (slice(None), pl.ds(off, hw))
        s_ref[sl] = jnp.sqrt(o[l][...].reshape(B, hw) * c[l][...].reshape(B, hw))
        lx_ref[sl] = locr[l][:, 0].reshape(B, hw)
        ly_ref[sl] = locr[l][:, 1].reshape(B, hw)
        sx_ref[sl] = sclr[l][:, 0].reshape(B, hw)
        sy_ref[sl] = sclr[l][:, 1].reshape(B, hw)


def _iota16():
    return lax.broadcasted_iota(jnp.int32, (16,), 0)


def _splat(x):
    return jnp.full((16,), x, dtype=jnp.int32)


def _sc_body(s_hbm, lx_hbm, ly_hbm, sx_hbm, sy_hbm,
             ocx, ocy, ow, oh, osc,
             sbuf, hist1, hist_p, cntbuf,
             cb_a, ci_a, cb_b, ci_b, offs, idxg,
             g0, g1, g2, g3, gs,
             sh_hist, sh_cb, sh_ci, sh_cnt, sem):
    cid = lax.axis_index("c")
    sid = lax.axis_index("s")
    row = cid * 8 + (sid >> 1)
    half = sid & 1

    iota = _iota16()
    zero16 = jnp.zeros((16,), jnp.int32)
    ones16 = jnp.ones((16,), jnp.int32)

    pltpu.sync_copy(s_hbm.at[row * 2 + half], sbuf)

    def z_body(i, c):
        hist1[pl.ds(i * 16, 16)] = zero16
        return c
    lax.fori_loop(0, H1 // 16, z_body, 0)

    def h_body(i, c):
        bits = plsc.bitcast(sbuf[pl.ds(i * 16, 16)], jnp.int32)
        d1 = bits >> 18
        cnt, last = plsc.scan_count(d1)
        plsc.addupdate_scatter(hist1, [d1], cnt, mask=last)
        return c
    lax.fori_loop(0, NV2, h_body, 0)

    pltpu.sync_copy(hist1, sh_hist.at[sid])
    plsc.subcore_barrier()
    pltpu.sync_copy(sh_hist.at[sid ^ 1], hist_p)

    def t_cond(st):
        j, cum, t1 = st
        return jnp.logical_and(cum < K, j >= 0)

    def t_body(st):
        j, cum, t1 = st
        v = hist1[pl.ds(j * 16, 16)] + hist_p[pl.ds(j * 16, 16)]
        rv = lax.rev(v, (0,))
        csum = plsc.cumsum(rv) + cum
        buckets = j * 16 + 15 - iota
        plsc.store_scatter(hist_p, [buckets], csum - rv)
        t1c = jnp.max(jnp.where(csum >= K, buckets, -1))
        t1 = jnp.maximum(t1, t1c)
        cum = cum + jnp.max(csum)
        return j - 1, cum, t1

    _, _, t1 = lax.while_loop(
        t_cond, t_body, (H1 // 16 - 1, jnp.int32(0), jnp.int32(-1)))

    ibase = half * N2

    def c_body(i, cnt_splat):
        bits = plsc.bitcast(sbuf[pl.ds(i * 16, 16)], jnp.int32)
        d1 = bits >> 18
        m = d1 >= t1
        pos = cnt_splat + plsc.cumsum(ones16, mask=m) - 1
        safe = jnp.logical_and(m, pos < CAPH)
        plsc.store_scatter(cb_a, [pos], bits, mask=safe)
        plsc.store_scatter(ci_a, [pos], ibase + i * 16 + iota, mask=safe)
        nm = plsc.all_reduce_population_count(m)
        return cnt_splat + nm
    cnt_splat = lax.fori_loop(0, NV2, c_body, zero16)
    cntbuf[pl.ds(0, 16)] = jnp.minimum(cnt_splat, CAPH)

    @pl.when(half == 1)
    def _():
        pltpu.sync_copy(cb_a.at[pl.ds(0, CAPH)], sh_cb.at[sid])
        pltpu.sync_copy(ci_a.at[pl.ds(0, CAPH)], sh_ci.at[sid])
        pltpu.sync_copy(cntbuf, sh_cnt.at[sid])
    plsc.subcore_barrier()

    @pl.when(half == 0)
    def _():
        n_a = jnp.max(cntbuf[pl.ds(0, 16)])
        pltpu.sync_copy(sh_cb.at[sid ^ 1], cb_a.at[pl.ds(CAPH, CAPH)])
        pltpu.sync_copy(sh_ci.at[sid ^ 1], ci_a.at[pl.ds(CAPH, CAPH)])
        pltpu.sync_copy(sh_cnt.at[sid ^ 1], cntbuf)
        n_b = jnp.max(cntbuf[pl.ds(0, 16)])
        csel = n_a + n_b
        nvc = (csel + 15) >> 4

        def rz_body(j, c):
            offs[pl.ds(j * 16, 16)] = zero16
            return c

        def rs_body(j, carry):
            v = offs[pl.ds(j * 16, 16)]
            csum = plsc.cumsum(v)
            offs[pl.ds(j * 16, 16)] = csum - v + carry
            return carry + jnp.full((16,), jnp.max(csum), jnp.int32)

        NOFS = 512 // 16

        lax.fori_loop(0, NOFS, rz_body, 0)

        def seg_hist(i, c, seg_off, seg_n):
            valid = i * 16 + iota < seg_n
            bits = cb_a[pl.ds(seg_off + i * 16, 16)]
            d = jnp.invert(bits) & 511
            cnt, last = plsc.scan_count(d, mask=valid)
            plsc.addupdate_scatter(offs, [d], cnt, mask=last)
            return c
        lax.fori_loop(0, (n_a + 15) >> 4,
                      functools.partial(seg_hist, seg_off=0, seg_n=n_a), 0)
        lax.fori_loop(0, (n_b + 15) >> 4,
                      functools.partial(seg_hist, seg_off=CAPH, seg_n=n_b), 0)

        lax.fori_loop(0, NOFS, rs_body, zero16)

        def seg_perm(i, c, seg_off, seg_n):
            valid = i * 16 + iota < seg_n
            bits = cb_a[pl.ds(seg_off + i * 16, 16)]
            idxv = ci_a[pl.ds(seg_off + i * 16, 16)]
            d = jnp.invert(bits) & 511
            cnt, last = plsc.scan_count(d, mask=valid)
            dest = plsc.load_gather(offs, [d]) + cnt - 1
            plsc.store_scatter(cb_b, [dest], bits, mask=valid)
            plsc.store_scatter(ci_b, [dest], idxv, mask=valid)
            plsc.addupdate_scatter(offs, [d], cnt, mask=last)
            return c
        lax.fori_loop(0, (n_a + 15) >> 4,
                      functools.partial(seg_perm, seg_off=0, seg_n=n_a), 0)
        lax.fori_loop(0, (n_b + 15) >> 4,
                      functools.partial(seg_perm, seg_off=CAPH, seg_n=n_b), 0)

        base = csel & ~15
        pad_m = jnp.logical_and(base + iota >= csel, base + iota < CAPT)
        plsc.store_scatter(cb_b, [base + iota], zero16, mask=pad_m)
        plsc.store_scatter(ci_b, [base + iota], _splat(N), mask=pad_m)

        lax.fori_loop(0, NOFS, rz_body, 0)

        def rh_body(i, c):
            bits = cb_b[pl.ds(i * 16, 16)]
            d = (jnp.invert(bits) >> 9) & 511
            cnt, last = plsc.scan_count(d)
            plsc.addupdate_scatter(offs, [d], cnt, mask=last)
            return c
        lax.fori_loop(0, nvc, rh_body, 0)

        lax.fori_loop(0, NOFS, rs_body, zero16)

        def rp_body(i, c):
            bits = cb_b[pl.ds(i * 16, 16)]
            idxv = ci_b[pl.ds(i * 16, 16)]
            d = (jnp.invert(bits) >> 9) & 511
            cnt, last = plsc.scan_count(d)
            dest = plsc.load_gather(offs, [d]) + cnt - 1
            plsc.store_scatter(cb_a, [dest], bits)
            plsc.store_scatter(ci_a, [dest], idxv)
            plsc.addupdate_scatter(offs, [d], cnt, mask=last)
            return c
        lax.fori_loop(0, nvc, rp_body, 0)

        def rf_body(i, c):
            bits = cb_a[pl.ds(i * 16, 16)]
            idxv = ci_a[pl.ds(i * 16, 16)]
            d = bits >> 18
            real = d >= t1
            cnt, last = plsc.scan_count(d, mask=real)
            dest = plsc.load_gather(hist_p, [d]) + cnt - 1
            safe = jnp.logical_and(real, dest < CAPT)
            plsc.store_scatter(cb_b, [dest], bits, mask=safe)
            plsc.store_scatter(ci_b, [dest], idxv, mask=safe)
            plsc.addupdate_scatter(hist_p, [d], cnt,
                                   mask=jnp.logical_and(last, real))
            return c
        lax.fori_loop(0, nvc, rf_body, 0)

        def g_body(i, c):
            idxg[pl.ds(i * 16, 16)] = ci_b[pl.ds(i * 16, 16)] + row * N
            return c
        lax.fori_loop(0, KV, g_body, 0)

        cp0 = pltpu.async_copy(lx_hbm.at[idxg], g0, sem)
        cp1 = pltpu.async_copy(ly_hbm.at[idxg], g1, sem)
        cp2 = pltpu.async_copy(sx_hbm.at[idxg], g2, sem)
        cp3 = pltpu.async_copy(sy_hbm.at[idxg], g3, sem)
        cp0.wait(); cp1.wait(); cp2.wait(); cp3.wait()

        def o_body(i, c):
            sl = pl.ds(i * 16, 16)
            idxv = ci_b[sl]
            bits = cb_b[sl]
            lvl = ((idxv >= OFFS[1]).astype(jnp.int32)
                   + (idxv >= OFFS[2]).astype(jnp.int32)
                   + (idxv >= OFFS[3]).astype(jnp.int32))
            off = jnp.where(lvl == 0, OFFS[0],
                  jnp.where(lvl == 1, OFFS[1],
                  jnp.where(lvl == 2, OFFS[2], OFFS[3])))
            log2w = 7 - lvl
            r = idxv - off
            x = (r & ((1 << log2w) - 1)).astype(jnp.float32)
            y = (r >> log2w).astype(jnp.float32)
            stf = (8 << lvl).astype(jnp.float32)
            g0[sl] = (x + 0.5) * stf + g0[sl] * stf
            g1[sl] = (y + 0.5) * stf + g1[sl] * stf
            g2[sl] = g2[sl] * stf
            g3[sl] = g3[sl] * stf
            gs[sl] = plsc.bitcast(bits, jnp.float32)
            return c
        lax.fori_loop(0, KV, o_body, 0)

        pltpu.sync_copy(g0, ocx.at[row])
        pltpu.sync_copy(g1, ocy.at[row])
        pltpu.sync_copy(g2, ow.at[row])
        pltpu.sync_copy(g3, oh.at[row])
        pltpu.sync_copy(gs, osc.at[row])


def kernel(objectness_0, centerness_0, location_0, scale_0,
           objectness_1, centerness_1, location_1, scale_1,
           objectness_2, centerness_2, location_2, scale_2,
           objectness_3, centerness_3, location_3, scale_3):
    objs = [objectness_0, objectness_1, objectness_2, objectness_3]
    ctrs = [centerness_0, centerness_1, centerness_2, centerness_3]
    locs = [location_0, location_1, location_2, location_3]
    scls = [scale_0, scale_1, scale_2, scale_3]

    s_all, lx2, ly2, sx2, sy2 = pl.pallas_call(
        _score_body,
        out_shape=[jax.ShapeDtypeStruct((B, N), jnp.float32)] * 5,
    )(*objs, *ctrs, *locs, *scls)
    lx, ly, sx, sy = (a.reshape(-1) for a in (lx2, ly2, sx2, sy2))

    mesh = plsc.VectorSubcoreMesh(core_axis_name="c", subcore_axis_name="s")
    out_t = [jax.ShapeDtypeStruct((B, K), jnp.float32)] * 5
    sc_topk = pl.kernel(
        _sc_body,
        out_type=out_t,
        mesh=mesh,
        compiler_params=pltpu.CompilerParams(needs_layout_passes=False),
        scratch_types=[
            pltpu.VMEM((N2,), jnp.float32),
            pltpu.VMEM((H1,), jnp.int32),
            pltpu.VMEM((H1,), jnp.int32),
            pltpu.VMEM((128,), jnp.int32),
            pltpu.VMEM((CAPT,), jnp.int32),
            pltpu.VMEM((CAPT,), jnp.int32),
            pltpu.VMEM((CAPT,), jnp.int32),
            pltpu.VMEM((CAPT,), jnp.int32),
            pltpu.VMEM((512,), jnp.int32),
            pltpu.VMEM((K,), jnp.int32),
            pltpu.VMEM((K,), jnp.float32),
            pltpu.VMEM((K,), jnp.float32),
            pltpu.VMEM((K,), jnp.float32),
            pltpu.VMEM((K,), jnp.float32),
            pltpu.VMEM((K,), jnp.float32),
            pltpu.VMEM_SHARED((16, H1), jnp.int32),
            pltpu.VMEM_SHARED((16, CAPH), jnp.int32),
            pltpu.VMEM_SHARED((16, CAPH), jnp.int32),
            pltpu.VMEM_SHARED((16, 128), jnp.int32),
            pltpu.SemaphoreType.DMA,
        ],
    )
    cx, cy, w, h, s = sc_topk(s_all.reshape(B * 2, N2), lx, ly, sx, sy)
    return jnp.stack([cx, cy, w, h, s], axis=-1)


# speedup vs baseline: 4.3210x; 1.0701x over previous
import functools

import jax
import jax.numpy as jnp
from jax import lax
from jax.experimental import pallas as pl
from jax.experimental.pallas import tpu as pltpu
from jax.experimental.pallas import tpu_sc as plsc

STRIDES = [8, 16, 32, 64]
LEVEL_HW = [(128, 128), (64, 64), (32, 32), (16, 16)]
OFFS = [0, 16384, 20480, 21504]
LOG2W = {128: 7, 64: 6, 32: 5, 16: 4}
N = 21760
N2 = N // 2
NV2 = N2 // 16
B = 16
K = 2000
KH = 1024
KL = K - KH
KP = 2 * KH
CAPH = 3072
CAPT = 2 * CAPH
H1 = 4096


def _score_body(*refs):
    o = refs[0:4]
    c = refs[4:8]
    locr = refs[8:12]
    sclr = refs[12:16]
    s_ref, lx_ref, ly_ref, sx_ref, sy_ref = refs[16:21]
    for l in range(4):
        off = OFFS[l]
        hw = LEVEL_HW[l][0] * LEVEL_HW[l][1]
        sl = (slice(None), pl.ds(off, hw))
        s_ref[sl] = jnp.sqrt(o[l][...].reshape(B, hw) * c[l][...].reshape(B, hw))
        lx_ref[sl] = locr[l][:, 0].reshape(B, hw)
        ly_ref[sl] = locr[l][:, 1].reshape(B, hw)
        sx_ref[sl] = sclr[l][:, 0].reshape(B, hw)
        sy_ref[sl] = sclr[l][:, 1].reshape(B, hw)


def _iota16():
    return lax.broadcasted_iota(jnp.int32, (16,), 0)


def _splat(x):
    return jnp.full((16,), x, dtype=jnp.int32)


def _sc_body(s_hbm, lx_hbm, ly_hbm, sx_hbm, sy_hbm,
             ocx, ocy, ow, oh, osc,
             sbuf, hist1, hist_p, cntbuf,
             cb_a, ci_a, cb_b, ci_b, offs, idxg,
             g0, g1, g2, g3, gs,
             sh_hist, sh_cb, sh_ci, sh_cnt, sem):
    cid = lax.axis_index("c")
    sid = lax.axis_index("s")
    row = cid * 8 + (sid >> 1)
    half = sid & 1

    iota = _iota16()
    zero16 = jnp.zeros((16,), jnp.int32)
    ones16 = jnp.ones((16,), jnp.int32)

    pltpu.sync_copy(s_hbm.at[row * 2 + half], sbuf)

    def z_body(i, c):
        hist1[pl.ds(i * 16, 16)] = zero16
        return c
    lax.fori_loop(0, H1 // 16, z_body, 0)

    def h_body(i, c):
        bits = plsc.bitcast(sbuf[pl.ds(i * 16, 16)], jnp.int32)
        d1 = bits >> 18
        cnt, last = plsc.scan_count(d1)
        plsc.addupdate_scatter(hist1, [d1], cnt, mask=last)
        return c
    lax.fori_loop(0, NV2, h_body, 0)

    pltpu.sync_copy(hist1, sh_hist.at[sid])
    plsc.subcore_barrier()
    pltpu.sync_copy(sh_hist.at[sid ^ 1], hist_p)

    def t_cond(st):
        j, cum, t1, t2, d_s = st
        return jnp.logical_and(cum < K, j >= 0)

    def t_body(st):
        j, cum, t1, t2, d_s = st
        v = hist1[pl.ds(j * 16, 16)] + hist_p[pl.ds(j * 16, 16)]
        rv = lax.rev(v, (0,))
        csum = plsc.cumsum(rv) + cum
        excl = csum - rv
        buckets = j * 16 + 15 - iota
        plsc.store_scatter(hist_p, [buckets], excl)
        t1 = jnp.maximum(t1, jnp.max(jnp.where(csum >= K, buckets, -1)))
        t2 = jnp.maximum(t2, jnp.max(jnp.where(csum > KH, buckets, -1)))
        d_s = jnp.minimum(d_s, jnp.min(jnp.where(excl < KH, buckets, H1)))
        cum = cum + jnp.max(csum)
        return j - 1, cum, t1, t2, d_s

    _, _, t1, t2, d_s = lax.while_loop(
        t_cond, t_body,
        (H1 // 16 - 1, jnp.int32(0), jnp.int32(-1), jnp.int32(-1),
         jnp.int32(H1)))

    ibase = half * N2

    def c_body(i, cnt_splat):
        bits = plsc.bitcast(sbuf[pl.ds(i * 16, 16)], jnp.int32)
        d1 = bits >> 18
        m = d1 >= t1
        pos = cnt_splat + plsc.cumsum(ones16, mask=m) - 1
        safe = jnp.logical_and(m, pos < CAPH)
        plsc.store_scatter(cb_a, [pos], bits, mask=safe)
        plsc.store_scatter(ci_a, [pos], ibase + i * 16 + iota, mask=safe)
        nm = plsc.all_reduce_population_count(m)
        return cnt_splat + nm
    cnt_splat = lax.fori_loop(0, NV2, c_body, zero16)
    n_own = jnp.max(jnp.minimum(cnt_splat, CAPH))

    def s_body(i, st):
        cK, cP = st
        sl = pl.ds(i * 16, 16)
        bits = cb_a[sl]
        idxv = ci_a[sl]
        valid = i * 16 + iota < n_own
        d = bits >> 18
        evm = jnp.logical_and(d >= d_s, valid)
        om = jnp.logical_and(d <= t2, valid)
        keepm = jnp.where(half == 0, evm, om)
        pubm = jnp.where(half == 0, om, evm)
        posK = cK + plsc.cumsum(ones16, mask=keepm) - 1
        posP = cP + plsc.cumsum(ones16, mask=pubm) - 1
        plsc.store_scatter(cb_b, [posK], bits,
                           mask=jnp.logical_and(keepm, posK < CAPH))
        plsc.store_scatter(ci_b, [posK], idxv,
                           mask=jnp.logical_and(keepm, posK < CAPH))
        plsc.store_scatter(cb_b, [CAPH + posP], bits,
                           mask=jnp.logical_and(pubm, posP < CAPH))
        plsc.store_scatter(ci_b, [CAPH + posP], idxv,
                           mask=jnp.logical_and(pubm, posP < CAPH))
        nk = plsc.all_reduce_population_count(keepm)
        npp = plsc.all_reduce_population_count(pubm)
        return cK + nk, cP + npp
    cK, cP = lax.fori_loop(0, (n_own + 15) >> 4, s_body, (zero16, zero16))
    n_keep = jnp.max(jnp.minimum(cK, CAPH))
    cntbuf[pl.ds(0, 16)] = jnp.minimum(cP, CAPH)

    pltpu.sync_copy(cb_b.at[pl.ds(CAPH, CAPH)], sh_cb.at[sid])
    pltpu.sync_copy(ci_b.at[pl.ds(CAPH, CAPH)], sh_ci.at[sid])
    pltpu.sync_copy(cntbuf, sh_cnt.at[sid])
    plsc.subcore_barrier()
    pltpu.sync_copy(sh_cb.at[sid ^ 1], cb_b.at[pl.ds(CAPH, CAPH)])
    pltpu.sync_copy(sh_ci.at[sid ^ 1], ci_b.at[pl.ds(CAPH, CAPH)])
    pltpu.sync_copy(sh_cnt.at[sid ^ 1], cntbuf)
    n_part = jnp.max(cntbuf[pl.ds(0, 16)])

    first_off = half * CAPH
    second_off = CAPH - first_off
    n_first = jnp.where(half == 0, n_keep, n_part)
    n_second = jnp.where(half == 0, n_part, n_keep)
    nt = n_first + n_second
    nvc = (nt + 15) >> 4

    def rz_body(j, c):
        offs[pl.ds(j * 16, 16)] = zero16
        return c

    def rs_body(j, carry):
        v = offs[pl.ds(j * 16, 16)]
        csum = plsc.cumsum(v)
        offs[pl.ds(j * 16, 16)] = csum - v + carry
        return carry + jnp.full((16,), jnp.max(csum), jnp.int32)

    NOFS = 512 // 16

    lax.fori_loop(0, NOFS, rz_body, 0)

    def seg_hist(i, c, seg_off, seg_n):
        valid = i * 16 + iota < seg_n
        bits = cb_b[pl.ds(seg_off + i * 16, 16)]
        d = jnp.invert(bits) & 511
        cnt, last = plsc.scan_count(d, mask=valid)
        plsc.addupdate_scatter(offs, [d], cnt, mask=last)
        return c
    lax.fori_loop(0, (n_first + 15) >> 4,
                  functools.partial(seg_hist, seg_off=first_off,
                                    seg_n=n_first), 0)
    lax.fori_loop(0, (n_second + 15) >> 4,
                  functools.partial(seg_hist, seg_off=second_off,
                                    seg_n=n_second), 0)

    lax.fori_loop(0, NOFS, rs_body, zero16)

    def seg_perm(i, c, seg_off, seg_n):
        valid = i * 16 + iota < seg_n
        bits = cb_b[pl.ds(seg_off + i * 16, 16)]
        idxv = ci_b[pl.ds(seg_off + i * 16, 16)]
        d = jnp.invert(bits) & 511
        cnt, last = plsc.scan_count(d, mask=valid)
        dest = plsc.load_gather(offs, [d]) + cnt - 1
        plsc.store_scatter(cb_a, [dest], bits, mask=valid)
        plsc.store_scatter(ci_a, [dest], idxv, mask=valid)
        plsc.addupdate_scatter(offs, [d], cnt, mask=last)
        return c
    lax.fori_loop(0, (n_first + 15) >> 4,
                  functools.partial(seg_perm, seg_off=first_off,
                                    seg_n=n_first), 0)
    lax.fori_loop(0, (n_second + 15) >> 4,
                  functools.partial(seg_perm, seg_off=second_off,
                                    seg_n=n_second), 0)

    base = nt & ~15
    pad_m = jnp.logical_and(base + iota >= nt, base + iota < CAPT)
    plsc.store_scatter(cb_a, [base + iota], zero16, mask=pad_m)
    plsc.store_scatter(ci_a, [base + iota], _splat(N), mask=pad_m)

    lax.fori_loop(0, NOFS, rz_body, 0)

    def rh_body(i, c):
        bits = cb_a[pl.ds(i * 16, 16)]
        d = (jnp.invert(bits) >> 9) & 511
        cnt, last = plsc.scan_count(d)
        plsc.addupdate_scatter(offs, [d], cnt, mask=last)
        return c
    lax.fori_loop(0, nvc, rh_body, 0)

    lax.fori_loop(0, NOFS, rs_body, zero16)

    def rp_body(i, c):
        bits = cb_a[pl.ds(i * 16, 16)]
        idxv = ci_a[pl.ds(i * 16, 16)]
        d = (jnp.invert(bits) >> 9) & 511
        cnt, last = plsc.scan_count(d)
        dest = plsc.load_gather(offs, [d]) + cnt - 1
        plsc.store_scatter(cb_b, [dest], bits)
        plsc.store_scatter(ci_b, [dest], idxv)
        plsc.addupdate_scatter(offs, [d], cnt, mask=last)
        return c
    lax.fori_loop(0, nvc, rp_body, 0)

    bo16 = plsc.load_gather(hist_p, [jnp.full((16,), t2, jnp.int32)])
    shift16 = jnp.where(half == 0, zero16, bo16)

    def rf_body(i, c):
        bits = cb_b[pl.ds(i * 16, 16)]
        idxv = ci_b[pl.ds(i * 16, 16)]
        d = bits >> 18
        real = d >= t1
        cnt, last = plsc.scan_count(d, mask=real)
        dest = plsc.load_gather(hist_p, [d]) + cnt - 1 - shift16
        safe = jnp.logical_and(real,
                               jnp.logical_and(dest >= 0, dest < CAPT))
        plsc.store_scatter(cb_a, [dest], bits, mask=safe)
        plsc.store_scatter(ci_a, [dest], idxv, mask=safe)
        plsc.addupdate_scatter(hist_p, [d], cnt,
                               mask=jnp.logical_and(last, real))
        return c
    lax.fori_loop(0, nvc, rf_body, 0)

    rdoff = jnp.where(half == 0, 0, KH - jnp.max(bo16))
    lim = jnp.where(half == 0, KH, KL)

    def g_body(i, c):
        ridx = rdoff + i * 16 + iota
        idxv = plsc.load_gather(ci_a, [ridx])
        valid = i * 16 + iota < lim
        idxg[pl.ds(i * 16, 16)] = jnp.where(valid, idxv + row * N, zero16)
        return c
    lax.fori_loop(0, KH // 16, g_body, 0)

    cp0 = pltpu.async_copy(lx_hbm.at[idxg], g0, sem)
    cp1 = pltpu.async_copy(ly_hbm.at[idxg], g1, sem)
    cp2 = pltpu.async_copy(sx_hbm.at[idxg], g2, sem)
    cp3 = pltpu.async_copy(sy_hbm.at[idxg], g3, sem)
    cp0.wait(); cp1.wait(); cp2.wait(); cp3.wait()

    def o_body(i, c):
        sl = pl.ds(i * 16, 16)
        ridx = rdoff + i * 16 + iota
        idxv = plsc.load_gather(ci_a, [ridx])
        bits = plsc.load_gather(cb_a, [ridx])
        lvl = ((idxv >= OFFS[1]).astype(jnp.int32)
               + (idxv >= OFFS[2]).astype(jnp.int32)
               + (idxv >= OFFS[3]).astype(jnp.int32))
        off = jnp.where(lvl == 0, OFFS[0],
              jnp.where(lvl == 1, OFFS[1],
              jnp.where(lvl == 2, OFFS[2], OFFS[3])))
        log2w = 7 - lvl
        r = idxv - off
        x = (r & ((1 << log2w) - 1)).astype(jnp.float32)
        y = (r >> log2w).astype(jnp.float32)
        stf = (8 << lvl).astype(jnp.float32)
        g0[sl] = (x + 0.5) * stf + g0[sl] * stf
        g1[sl] = (y + 0.5) * stf + g1[sl] * stf
        g2[sl] = g2[sl] * stf
        g3[sl] = g3[sl] * stf
        gs[sl] = plsc.bitcast(bits, jnp.float32)
        return c
    lax.fori_loop(0, KH // 16, o_body, 0)

    @pl.when(half == 0)
    def _():
        pltpu.sync_copy(g0, ocx.at[row, pl.ds(0, KH)])
        pltpu.sync_copy(g1, ocy.at[row, pl.ds(0, KH)])
        pltpu.sync_copy(g2, ow.at[row, pl.ds(0, KH)])
        pltpu.sync_copy(g3, oh.at[row, pl.ds(0, KH)])
        pltpu.sync_copy(gs, osc.at[row, pl.ds(0, KH)])

    @pl.when(half == 1)
    def _():
        pltpu.sync_copy(g0, ocx.at[row, pl.ds(KH, KH)])
        pltpu.sync_copy(g1, ocy.at[row, pl.ds(KH, KH)])
        pltpu.sync_copy(g2, ow.at[row, pl.ds(KH, KH)])
        pltpu.sync_copy(g3, oh.at[row, pl.ds(KH, KH)])
        pltpu.sync_copy(gs, osc.at[row, pl.ds(KH, KH)])


def kernel(objectness_0, centerness_0, location_0, scale_0,
           objectness_1, centerness_1, location_1, scale_1,
           objectness_2, centerness_2, location_2, scale_2,
           objectness_3, centerness_3, location_3, scale_3):
    objs = [objectness_0, objectness_1, objectness_2, objectness_3]
    ctrs = [centerness_0, centerness_1, centerness_2, centerness_3]
    locs = [location_0, location_1, location_2, location_3]
    scls = [scale_0, scale_1, scale_2, scale_3]

    s_all, lx2, ly2, sx2, sy2 = pl.pallas_call(
        _score_body,
        out_shape=[jax.ShapeDtypeStruct((B, N), jnp.float32)] * 5,
    )(*objs, *ctrs, *locs, *scls)
    lx, ly, sx, sy = (a.reshape(-1) for a in (lx2, ly2, sx2, sy2))

    mesh = plsc.VectorSubcoreMesh(core_axis_name="c", subcore_axis_name="s")
    out_t = [jax.ShapeDtypeStruct((B, KP), jnp.float32)] * 5
    sc_topk = pl.kernel(
        _sc_body,
        out_type=out_t,
        mesh=mesh,
        compiler_params=pltpu.CompilerParams(needs_layout_passes=False),
        scratch_types=[
            pltpu.VMEM((N2,), jnp.float32),
            pltpu.VMEM((H1,), jnp.int32),
            pltpu.VMEM((H1,), jnp.int32),
            pltpu.VMEM((128,), jnp.int32),
            pltpu.VMEM((CAPT,), jnp.int32),
            pltpu.VMEM((CAPT,), jnp.int32),
            pltpu.VMEM((CAPT,), jnp.int32),
            pltpu.VMEM((CAPT,), jnp.int32),
            pltpu.VMEM((512,), jnp.int32),
            pltpu.VMEM((KH,), jnp.int32),
            pltpu.VMEM((KH,), jnp.float32),
            pltpu.VMEM((KH,), jnp.float32),
            pltpu.VMEM((KH,), jnp.float32),
            pltpu.VMEM((KH,), jnp.float32),
            pltpu.VMEM((KH,), jnp.float32),
            pltpu.VMEM_SHARED((16, H1), jnp.int32),
            pltpu.VMEM_SHARED((16, CAPH), jnp.int32),
            pltpu.VMEM_SHARED((16, CAPH), jnp.int32),
            pltpu.VMEM_SHARED((16, 128), jnp.int32),
            pltpu.SemaphoreType.DMA,
        ],
    )
    cx, cy, w, h, s = sc_topk(s_all.reshape(B * 2, N2), lx, ly, sx, sy)
    return jnp.stack([cx, cy, w, h, s], axis=-1)[:, :K, :]


# speedup vs baseline: 4.4058x; 1.0196x over previous
import functools

import jax
import jax.numpy as jnp
from jax import lax
from jax.experimental import pallas as pl
from jax.experimental.pallas import tpu as pltpu
from jax.experimental.pallas import tpu_sc as plsc

STRIDES = [8, 16, 32, 64]
LEVEL_HW = [(128, 128), (64, 64), (32, 32), (16, 16)]
OFFS = [0, 16384, 20480, 21504]
LOG2W = {128: 7, 64: 6, 32: 5, 16: 4}
N = 21760
N2 = N // 2
NV2 = N2 // 16
B = 16
K = 2000
KH = 1024
KL = K - KH
KP = 2 * KH
CAPH = 3072
CAPT = 2 * CAPH
H1 = 4096


def _score_body(*refs):
    o = refs[0:4]
    c = refs[4:8]
    locr = refs[8:12]
    sclr = refs[12:16]
    s_ref, lx_ref, ly_ref, sx_ref, sy_ref = refs[16:21]
    for l in range(4):
        off = OFFS[l]
        hw = LEVEL_HW[l][0] * LEVEL_HW[l][1]
        sl = (slice(None), pl.ds(off, hw))
        s_ref[sl] = jnp.sqrt(o[l][...].reshape(B, hw) * c[l][...].reshape(B, hw))
        lx_ref[sl] = locr[l][:, 0].reshape(B, hw)
        ly_ref[sl] = locr[l][:, 1].reshape(B, hw)
        sx_ref[sl] = sclr[l][:, 0].reshape(B, hw)
        sy_ref[sl] = sclr[l][:, 1].reshape(B, hw)


def _iota16():
    return lax.broadcasted_iota(jnp.int32, (16,), 0)


def _splat(x):
    return jnp.full((16,), x, dtype=jnp.int32)


def _sc_body(s_hbm, lx_hbm, ly_hbm, sx_hbm, sy_hbm,
             ocx, ocy, ow, oh, osc,
             sbuf, hist1, hist_p, cntbuf,
             cb_a, ci_a, cb_b, ci_b, offs, idxg,
             g0, g1, g2, g3, gs,
             sh_hist, sh_cb, sh_ci, sh_cnt, sem):
    cid = lax.axis_index("c")
    sid = lax.axis_index("s")
    row = cid * 8 + (sid >> 1)
    half = sid & 1

    iota = _iota16()
    zero16 = jnp.zeros((16,), jnp.int32)
    ones16 = jnp.ones((16,), jnp.int32)

    pltpu.sync_copy(s_hbm.at[row * 2 + half], sbuf)

    def z_body(i, c):
        hist1[pl.ds(i * 16, 16)] = zero16
        return c
    lax.fori_loop(0, H1 // 16, z_body, 0)

    def h_body(i, c):
        for u in range(4):
            bits = plsc.bitcast(sbuf[pl.ds((i * 4 + u) * 16, 16)], jnp.int32)
            d1 = bits >> 18
            cnt, last = plsc.scan_count(d1)
            plsc.addupdate_scatter(hist1, [d1], cnt, mask=last)
        return c
    lax.fori_loop(0, NV2 // 4, h_body, 0)

    pltpu.sync_copy(hist1, sh_hist.at[sid])
    plsc.subcore_barrier()
    pltpu.sync_copy(sh_hist.at[sid ^ 1], hist_p)

    def t_cond(st):
        j, cum, t1, t2, d_s = st
        return jnp.logical_and(cum < K, j >= 0)

    def t_body(st):
        j, cum, t1, t2, d_s = st
        v = hist1[pl.ds(j * 16, 16)] + hist_p[pl.ds(j * 16, 16)]
        rv = lax.rev(v, (0,))
        csum = plsc.cumsum(rv) + cum
        excl = csum - rv
        buckets = j * 16 + 15 - iota
        plsc.store_scatter(hist_p, [buckets], excl)
        t1 = jnp.maximum(t1, jnp.max(jnp.where(csum >= K, buckets, -1)))
        t2 = jnp.maximum(t2, jnp.max(jnp.where(csum > KH, buckets, -1)))
        d_s = jnp.minimum(d_s, jnp.min(jnp.where(excl < KH, buckets, H1)))
        cum = cum + jnp.max(csum)
        return j - 1, cum, t1, t2, d_s

    _, _, t1, t2, d_s = lax.while_loop(
        t_cond, t_body,
        (H1 // 16 - 1, jnp.int32(0), jnp.int32(-1), jnp.int32(-1),
         jnp.int32(H1)))

    ibase = half * N2

    def c_body(i, cnt_splat):
        for u in range(4):
            k = i * 4 + u
            bits = plsc.bitcast(sbuf[pl.ds(k * 16, 16)], jnp.int32)
            d1 = bits >> 18
            m = d1 >= t1
            pos = cnt_splat + plsc.cumsum(ones16, mask=m) - 1
            safe = jnp.logical_and(m, pos < CAPH)
            plsc.store_scatter(cb_a, [pos], bits, mask=safe)
            plsc.store_scatter(ci_a, [pos], ibase + k * 16 + iota, mask=safe)
            nm = plsc.all_reduce_population_count(m)
            cnt_splat = cnt_splat + nm
        return cnt_splat
    cnt_splat = lax.fori_loop(0, NV2 // 4, c_body, zero16)
    n_own = jnp.max(jnp.minimum(cnt_splat, CAPH))

    def s_body(i, st):
        cK, cP = st
        sl = pl.ds(i * 16, 16)
        bits = cb_a[sl]
        idxv = ci_a[sl]
        valid = i * 16 + iota < n_own
        d = bits >> 18
        evm = jnp.logical_and(d >= d_s, valid)
        om = jnp.logical_and(d <= t2, valid)
        keepm = jnp.where(half == 0, evm, om)
        pubm = jnp.where(half == 0, om, evm)
        posK = cK + plsc.cumsum(ones16, mask=keepm) - 1
        posP = cP + plsc.cumsum(ones16, mask=pubm) - 1
        plsc.store_scatter(cb_b, [posK], bits,
                           mask=jnp.logical_and(keepm, posK < CAPH))
        plsc.store_scatter(ci_b, [posK], idxv,
                           mask=jnp.logical_and(keepm, posK < CAPH))
        plsc.store_scatter(cb_b, [CAPH + posP], bits,
                           mask=jnp.logical_and(pubm, posP < CAPH))
        plsc.store_scatter(ci_b, [CAPH + posP], idxv,
                           mask=jnp.logical_and(pubm, posP < CAPH))
        nk = plsc.all_reduce_population_count(keepm)
        npp = plsc.all_reduce_population_count(pubm)
        return cK + nk, cP + npp
    cK, cP = lax.fori_loop(0, (n_own + 15) >> 4, s_body, (zero16, zero16))
    n_keep = jnp.max(jnp.minimum(cK, CAPH))
    cntbuf[pl.ds(0, 16)] = jnp.minimum(cP, CAPH)

    pltpu.sync_copy(cb_b.at[pl.ds(CAPH, CAPH)], sh_cb.at[sid])
    pltpu.sync_copy(ci_b.at[pl.ds(CAPH, CAPH)], sh_ci.at[sid])
    pltpu.sync_copy(cntbuf, sh_cnt.at[sid])
    plsc.subcore_barrier()
    pltpu.sync_copy(sh_cb.at[sid ^ 1], cb_b.at[pl.ds(CAPH, CAPH)])
    pltpu.sync_copy(sh_ci.at[sid ^ 1], ci_b.at[pl.ds(CAPH, CAPH)])
    pltpu.sync_copy(sh_cnt.at[sid ^ 1], cntbuf)
    n_part = jnp.max(cntbuf[pl.ds(0, 16)])

    first_off = half * CAPH
    second_off = CAPH - first_off
    n_first = jnp.where(half == 0, n_keep, n_part)
    n_second = jnp.where(half == 0, n_part, n_keep)
    nt = n_first + n_second
    nvc = (nt + 15) >> 4

    def rz_body(j, c):
        offs[pl.ds(j * 16, 16)] = zero16
        return c

    def rs_body(j, carry):
        v = offs[pl.ds(j * 16, 16)]
        csum = plsc.cumsum(v)
        offs[pl.ds(j * 16, 16)] = csum - v + carry
        return carry + jnp.full((16,), jnp.max(csum), jnp.int32)

    NOFS = 512 // 16

    lax.fori_loop(0, NOFS, rz_body, 0)

    def seg_hist(i, c, seg_off, seg_n):
        valid = i * 16 + iota < seg_n
        bits = cb_b[pl.ds(seg_off + i * 16, 16)]
        d = jnp.invert(bits) & 511
        cnt, last = plsc.scan_count(d, mask=valid)
        plsc.addupdate_scatter(offs, [d], cnt, mask=last)
        return c
    lax.fori_loop(0, (n_first + 15) >> 4,
                  functools.partial(seg_hist, seg_off=first_off,
                                    seg_n=n_first), 0)
    lax.fori_loop(0, (n_second + 15) >> 4,
                  functools.partial(seg_hist, seg_off=second_off,
                                    seg_n=n_second), 0)

    lax.fori_loop(0, NOFS, rs_body, zero16)

    def seg_perm(i, c, seg_off, seg_n):
        valid = i * 16 + iota < seg_n
        bits = cb_b[pl.ds(seg_off + i * 16, 16)]
        idxv = ci_b[pl.ds(seg_off + i * 16, 16)]
        d = jnp.invert(bits) & 511
        cnt, last = plsc.scan_count(d, mask=valid)
        dest = plsc.load_gather(offs, [d]) + cnt - 1
        plsc.store_scatter(cb_a, [dest], bits, mask=valid)
        plsc.store_scatter(ci_a, [dest], idxv, mask=valid)
        plsc.addupdate_scatter(offs, [d], cnt, mask=last)
        return c
    lax.fori_loop(0, (n_first + 15) >> 4,
                  functools.partial(seg_perm, seg_off=first_off,
                                    seg_n=n_first), 0)
    lax.fori_loop(0, (n_second + 15) >> 4,
                  functools.partial(seg_perm, seg_off=second_off,
                                    seg_n=n_second), 0)

    base = nt & ~15
    pad_m = jnp.logical_and(base + iota >= nt, base + iota < CAPT)
    plsc.store_scatter(cb_a, [base + iota], zero16, mask=pad_m)
    plsc.store_scatter(ci_a, [base + iota], _splat(N), mask=pad_m)

    lax.fori_loop(0, NOFS, rz_body, 0)

    def rh_body(i, c):
        bits = cb_a[pl.ds(i * 16, 16)]
        d = (jnp.invert(bits) >> 9) & 511
        cnt, last = plsc.scan_count(d)
        plsc.addupdate_scatter(offs, [d], cnt, mask=last)
        return c
    lax.fori_loop(0, nvc, rh_body, 0)

    lax.fori_loop(0, NOFS, rs_body, zero16)

    def rp_body(i, c):
        bits = cb_a[pl.ds(i * 16, 16)]
        idxv = ci_a[pl.ds(i * 16, 16)]
        d = (jnp.invert(bits) >> 9) & 511
        cnt, last = plsc.scan_count(d)
        dest = plsc.load_gather(offs, [d]) + cnt - 1
        plsc.store_scatter(cb_b, [dest], bits)
        plsc.store_scatter(ci_b, [dest], idxv)
        plsc.addupdate_scatter(offs, [d], cnt, mask=last)
        return c
    lax.fori_loop(0, nvc, rp_body, 0)

    bo16 = plsc.load_gather(hist_p, [jnp.full((16,), t2, jnp.int32)])
    shift16 = jnp.where(half == 0, zero16, bo16)

    def rf_body(i, c):
        bits = cb_b[pl.ds(i * 16, 16)]
        idxv = ci_b[pl.ds(i * 16, 16)]
        d = bits >> 18
        real = d >= t1
        cnt, last = plsc.scan_count(d, mask=real)
        dest = plsc.load_gather(hist_p, [d]) + cnt - 1 - shift16
        safe = jnp.logical_and(real,
                               jnp.logical_and(dest >= 0, dest < CAPT))
        plsc.store_scatter(cb_a, [dest], bits, mask=safe)
        plsc.store_scatter(ci_a, [dest], idxv, mask=safe)
        plsc.addupdate_scatter(hist_p, [d], cnt,
                               mask=jnp.logical_and(last, real))
        return c
    lax.fori_loop(0, nvc, rf_body, 0)

    rdoff = jnp.where(half == 0, 0, KH - jnp.max(bo16))
    lim = jnp.where(half == 0, KH, KL)

    def g_body(i, c):
        ridx = rdoff + i * 16 + iota
        idxv = plsc.load_gather(ci_a, [ridx])
        valid = i * 16 + iota < lim
        idxg[pl.ds(i * 16, 16)] = jnp.where(valid, idxv + row * N, zero16)
        return c
    lax.fori_loop(0, KH // 16, g_body, 0)

    cp0 = pltpu.async_copy(lx_hbm.at[idxg], g0, sem)
    cp1 = pltpu.async_copy(ly_hbm.at[idxg], g1, sem)
    cp2 = pltpu.async_copy(sx_hbm.at[idxg], g2, sem)
    cp3 = pltpu.async_copy(sy_hbm.at[idxg], g3, sem)
    cp0.wait(); cp1.wait(); cp2.wait(); cp3.wait()

    def o_body(i, c):
        sl = pl.ds(i * 16, 16)
        ridx = rdoff + i * 16 + iota
        idxv = plsc.load_gather(ci_a, [ridx])
        bits = plsc.load_gather(cb_a, [ridx])
        lvl = ((idxv >= OFFS[1]).astype(jnp.int32)
               + (idxv >= OFFS[2]).astype(jnp.int32)
               + (idxv >= OFFS[3]).astype(jnp.int32))
        off = jnp.where(lvl == 0, OFFS[0],
              jnp.where(lvl == 1, OFFS[1],
              jnp.where(lvl == 2, OFFS[2], OFFS[3])))
        log2w = 7 - lvl
        r = idxv - off
        x = (r & ((1 << log2w) - 1)).astype(jnp.float32)
        y = (r >> log2w).astype(jnp.float32)
        stf = (8 << lvl).astype(jnp.float32)
        g0[sl] = (x + 0.5) * stf + g0[sl] * stf
        g1[sl] = (y + 0.5) * stf + g1[sl] * stf
        g2[sl] = g2[sl] * stf
        g3[sl] = g3[sl] * stf
        gs[sl] = plsc.bitcast(bits, jnp.float32)
        return c
    lax.fori_loop(0, KH // 16, o_body, 0)

    @pl.when(half == 0)
    def _():
        pltpu.sync_copy(g0, ocx.at[row, pl.ds(0, KH)])
        pltpu.sync_copy(g1, ocy.at[row, pl.ds(0, KH)])
        pltpu.sync_copy(g2, ow.at[row, pl.ds(0, KH)])
        pltpu.sync_copy(g3, oh.at[row, pl.ds(0, KH)])
        pltpu.sync_copy(gs, osc.at[row, pl.ds(0, KH)])

    @pl.when(half == 1)
    def _():
        pltpu.sync_copy(g0, ocx.at[row, pl.ds(KH, KH)])
        pltpu.sync_copy(g1, ocy.at[row, pl.ds(KH, KH)])
        pltpu.sync_copy(g2, ow.at[row, pl.ds(KH, KH)])
        pltpu.sync_copy(g3, oh.at[row, pl.ds(KH, KH)])
        pltpu.sync_copy(gs, osc.at[row, pl.ds(KH, KH)])


def kernel(objectness_0, centerness_0, location_0, scale_0,
           objectness_1, centerness_1, location_1, scale_1,
           objectness_2, centerness_2, location_2, scale_2,
           objectness_3, centerness_3, location_3, scale_3):
    objs = [objectness_0, objectness_1, objectness_2, objectness_3]
    ctrs = [centerness_0, centerness_1, centerness_2, centerness_3]
    locs = [location_0, location_1, location_2, location_3]
    scls = [scale_0, scale_1, scale_2, scale_3]

    s_all, lx2, ly2, sx2, sy2 = pl.pallas_call(
        _score_body,
        out_shape=[jax.ShapeDtypeStruct((B, N), jnp.float32)] * 5,
    )(*objs, *ctrs, *locs, *scls)
    lx, ly, sx, sy = (a.reshape(-1) for a in (lx2, ly2, sx2, sy2))

    mesh = plsc.VectorSubcoreMesh(core_axis_name="c", subcore_axis_name="s")
    out_t = [jax.ShapeDtypeStruct((B, KP), jnp.float32)] * 5
    sc_topk = pl.kernel(
        _sc_body,
        out_type=out_t,
        mesh=mesh,
        compiler_params=pltpu.CompilerParams(needs_layout_passes=False),
        scratch_types=[
            pltpu.VMEM((N2,), jnp.float32),
            pltpu.VMEM((H1,), jnp.int32),
            pltpu.VMEM((H1,), jnp.int32),
            pltpu.VMEM((128,), jnp.int32),
            pltpu.VMEM((CAPT,), jnp.int32),
            pltpu.VMEM((CAPT,), jnp.int32),
            pltpu.VMEM((CAPT,), jnp.int32),
            pltpu.VMEM((CAPT,), jnp.int32),
            pltpu.VMEM((512,), jnp.int32),
            pltpu.VMEM((KH,), jnp.int32),
            pltpu.VMEM((KH,), jnp.float32),
            pltpu.VMEM((KH,), jnp.float32),
            pltpu.VMEM((KH,), jnp.float32),
            pltpu.VMEM((KH,), jnp.float32),
            pltpu.VMEM((KH,), jnp.float32),
            pltpu.VMEM_SHARED((16, H1), jnp.int32),
            pltpu.VMEM_SHARED((16, CAPH), jnp.int32),
            pltpu.VMEM_SHARED((16, CAPH), jnp.int32),
            pltpu.VMEM_SHARED((16, 128), jnp.int32),
            pltpu.SemaphoreType.DMA,
        ],
    )
    cx, cy, w, h, s = sc_topk(s_all.reshape(B * 2, N2), lx, ly, sx, sy)
    return jnp.stack([cx, cy, w, h, s], axis=-1)[:, :K, :]


# speedup vs baseline: 4.4423x; 1.0083x over previous
import functools

import jax
import jax.numpy as jnp
from jax import lax
from jax.experimental import pallas as pl
from jax.experimental.pallas import tpu as pltpu
from jax.experimental.pallas import tpu_sc as plsc

STRIDES = [8, 16, 32, 64]
LEVEL_HW = [(128, 128), (64, 64), (32, 32), (16, 16)]
OFFS = [0, 16384, 20480, 21504]
LOG2W = {128: 7, 64: 6, 32: 5, 16: 4}
N = 21760
N2 = N // 2
NV2 = N2 // 16
B = 16
K = 2000
KH = 1024
KL = K - KH
KP = 2 * KH
CAPH = 3072
CAPT = 2 * CAPH
H1 = 4096


def _score_body(*refs):
    o = refs[0:4]
    c = refs[4:8]
    locr = refs[8:12]
    sclr = refs[12:16]
    s_ref, lx_ref, ly_ref, sx_ref, sy_ref = refs[16:21]
    for l in range(4):
        off = OFFS[l]
        hw = LEVEL_HW[l][0] * LEVEL_HW[l][1]
        sl = (slice(None), pl.ds(off, hw))
        s_ref[sl] = jnp.sqrt(o[l][...].reshape(B, hw) * c[l][...].reshape(B, hw))
        lx_ref[sl] = locr[l][:, 0].reshape(B, hw)
        ly_ref[sl] = locr[l][:, 1].reshape(B, hw)
        sx_ref[sl] = sclr[l][:, 0].reshape(B, hw)
        sy_ref[sl] = sclr[l][:, 1].reshape(B, hw)


def _iota16():
    return lax.broadcasted_iota(jnp.int32, (16,), 0)


def _splat(x):
    return jnp.full((16,), x, dtype=jnp.int32)


def _sc_body(s_hbm, lx_hbm, ly_hbm, sx_hbm, sy_hbm,
             ocx, ocy, ow, oh, osc,
             sbuf, hist1, hist_p, cntbuf,
             cb_a, ci_a, cb_b, ci_b, offs, idxg,
             g0, g1, g2, g3, gs,
             sh_hist, sh_cb, sh_ci, sh_cnt, sem):
    cid = lax.axis_index("c")
    sid = lax.axis_index("s")
    row = cid * 8 + (sid >> 1)
    half = sid & 1

    iota = _iota16()
    zero16 = jnp.zeros((16,), jnp.int32)
    ones16 = jnp.ones((16,), jnp.int32)

    pltpu.sync_copy(s_hbm.at[row * 2 + half], sbuf)

    def z_body(i, c):
        for u in range(8):
            hist1[pl.ds((i * 8 + u) * 16, 16)] = zero16
        return c
    lax.fori_loop(0, H1 // 128, z_body, 0)

    def h_body(i, c):
        for u in range(4):
            bits = plsc.bitcast(sbuf[pl.ds((i * 4 + u) * 16, 16)], jnp.int32)
            d1 = bits >> 18
            cnt, last = plsc.scan_count(d1)
            plsc.addupdate_scatter(hist1, [d1], cnt, mask=last)
        return c
    lax.fori_loop(0, NV2 // 4, h_body, 0)

    pltpu.sync_copy(hist1, sh_hist.at[sid])
    plsc.subcore_barrier()
    pltpu.sync_copy(sh_hist.at[sid ^ 1], hist_p)

    def t_cond(st):
        j, cum, t1, t2, d_s = st
        return jnp.logical_and(cum < K, j >= 0)

    def t_body(st):
        j, cum, t1, t2, d_s = st
        v = hist1[pl.ds(j * 16, 16)] + hist_p[pl.ds(j * 16, 16)]
        rv = lax.rev(v, (0,))
        csum = plsc.cumsum(rv) + cum
        excl = csum - rv
        buckets = j * 16 + 15 - iota
        plsc.store_scatter(hist_p, [buckets], excl)
        t1 = jnp.maximum(t1, jnp.max(jnp.where(csum >= K, buckets, -1)))
        t2 = jnp.maximum(t2, jnp.max(jnp.where(csum > KH, buckets, -1)))
        d_s = jnp.minimum(d_s, jnp.min(jnp.where(excl < KH, buckets, H1)))
        cum = cum + jnp.max(csum)
        return j - 1, cum, t1, t2, d_s

    _, _, t1, t2, d_s = lax.while_loop(
        t_cond, t_body,
        (H1 // 16 - 1, jnp.int32(0), jnp.int32(-1), jnp.int32(-1),
         jnp.int32(H1)))

    ibase = half * N2

    def c_body(i, cnt_splat):
        for u in range(4):
            k = i * 4 + u
            bits = plsc.bitcast(sbuf[pl.ds(k * 16, 16)], jnp.int32)
            d1 = bits >> 18
            m = d1 >= t1
            pos = cnt_splat + plsc.cumsum(ones16, mask=m) - 1
            safe = jnp.logical_and(m, pos < CAPH)
            plsc.store_scatter(cb_a, [pos], bits, mask=safe)
            plsc.store_scatter(ci_a, [pos], ibase + k * 16 + iota, mask=safe)
            nm = plsc.all_reduce_population_count(m)
            cnt_splat = cnt_splat + nm
        return cnt_splat
    cnt_splat = lax.fori_loop(0, NV2 // 4, c_body, zero16)
    n_own = jnp.max(jnp.minimum(cnt_splat, CAPH))

    def s_body(i, st):
        cK, cP = st
        sl = pl.ds(i * 16, 16)
        bits = cb_a[sl]
        idxv = ci_a[sl]
        valid = i * 16 + iota < n_own
        d = bits >> 18
        evm = jnp.logical_and(d >= d_s, valid)
        om = jnp.logical_and(d <= t2, valid)
        keepm = jnp.where(half == 0, evm, om)
        pubm = jnp.where(half == 0, om, evm)
        posK = cK + plsc.cumsum(ones16, mask=keepm) - 1
        posP = cP + plsc.cumsum(ones16, mask=pubm) - 1
        plsc.store_scatter(cb_b, [posK], bits,
                           mask=jnp.logical_and(keepm, posK < CAPH))
        plsc.store_scatter(ci_b, [posK], idxv,
                           mask=jnp.logical_and(keepm, posK < CAPH))
        plsc.store_scatter(cb_b, [CAPH + posP], bits,
                           mask=jnp.logical_and(pubm, posP < CAPH))
        plsc.store_scatter(ci_b, [CAPH + posP], idxv,
                           mask=jnp.logical_and(pubm, posP < CAPH))
        nk = plsc.all_reduce_population_count(keepm)
        npp = plsc.all_reduce_population_count(pubm)
        return cK + nk, cP + npp
    cK, cP = lax.fori_loop(0, (n_own + 15) >> 4, s_body, (zero16, zero16))
    n_keep = jnp.max(jnp.minimum(cK, CAPH))
    cntbuf[pl.ds(0, 16)] = jnp.minimum(cP, CAPH)

    pltpu.sync_copy(cb_b.at[pl.ds(CAPH, CAPH)], sh_cb.at[sid])
    pltpu.sync_copy(ci_b.at[pl.ds(CAPH, CAPH)], sh_ci.at[sid])
    pltpu.sync_copy(cntbuf, sh_cnt.at[sid])
    plsc.subcore_barrier()
    pltpu.sync_copy(sh_cb.at[sid ^ 1], cb_b.at[pl.ds(CAPH, CAPH)])
    pltpu.sync_copy(sh_ci.at[sid ^ 1], ci_b.at[pl.ds(CAPH, CAPH)])
    pltpu.sync_copy(sh_cnt.at[sid ^ 1], cntbuf)
    n_part = jnp.max(cntbuf[pl.ds(0, 16)])

    first_off = half * CAPH
    second_off = CAPH - first_off
    n_first = jnp.where(half == 0, n_keep, n_part)
    n_second = jnp.where(half == 0, n_part, n_keep)
    nt = n_first + n_second
    nvc = (nt + 15) >> 4

    def rz_body(j, c):
        offs[pl.ds(j * 16, 16)] = zero16
        return c

    def rs_body(j, carry):
        v = offs[pl.ds(j * 16, 16)]
        csum = plsc.cumsum(v)
        offs[pl.ds(j * 16, 16)] = csum - v + carry
        return carry + jnp.full((16,), jnp.max(csum), jnp.int32)

    NOFS = 512 // 16

    lax.fori_loop(0, NOFS, rz_body, 0)

    def seg_hist(i, c, seg_off, seg_n):
        valid = i * 16 + iota < seg_n
        bits = cb_b[pl.ds(seg_off + i * 16, 16)]
        d = jnp.invert(bits) & 511
        cnt, last = plsc.scan_count(d, mask=valid)
        plsc.addupdate_scatter(offs, [d], cnt, mask=last)
        return c
    lax.fori_loop(0, (n_first + 15) >> 4,
                  functools.partial(seg_hist, seg_off=first_off,
                                    seg_n=n_first), 0)
    lax.fori_loop(0, (n_second + 15) >> 4,
                  functools.partial(seg_hist, seg_off=second_off,
                                    seg_n=n_second), 0)

    lax.fori_loop(0, NOFS, rs_body, zero16)

    def seg_perm(i, c, seg_off, seg_n):
        valid = i * 16 + iota < seg_n
        bits = cb_b[pl.ds(seg_off + i * 16, 16)]
        idxv = ci_b[pl.ds(seg_off + i * 16, 16)]
        d = jnp.invert(bits) & 511
        cnt, last = plsc.scan_count(d, mask=valid)
        dest = plsc.load_gather(offs, [d]) + cnt - 1
        plsc.store_scatter(cb_a, [dest], bits, mask=valid)
        plsc.store_scatter(ci_a, [dest], idxv, mask=valid)
        plsc.addupdate_scatter(offs, [d], cnt, mask=last)
        return c
    lax.fori_loop(0, (n_first + 15) >> 4,
                  functools.partial(seg_perm, seg_off=first_off,
                                    seg_n=n_first), 0)
    lax.fori_loop(0, (n_second + 15) >> 4,
                  functools.partial(seg_perm, seg_off=second_off,
                                    seg_n=n_second), 0)

    base = nt & ~15
    pad_m = jnp.logical_and(base + iota >= nt, base + iota < CAPT)
    plsc.store_scatter(cb_a, [base + iota], zero16, mask=pad_m)
    plsc.store_scatter(ci_a, [base + iota], _splat(N), mask=pad_m)

    lax.fori_loop(0, NOFS, rz_body, 0)

    def rh_body(i, c):
        bits = cb_a[pl.ds(i * 16, 16)]
        d = (jnp.invert(bits) >> 9) & 511
        cnt, last = plsc.scan_count(d)
        plsc.addupdate_scatter(offs, [d], cnt, mask=last)
        return c
    lax.fori_loop(0, nvc, rh_body, 0)

    lax.fori_loop(0, NOFS, rs_body, zero16)

    def rp_body(i, c):
        bits = cb_a[pl.ds(i * 16, 16)]
        idxv = ci_a[pl.ds(i * 16, 16)]
        d = (jnp.invert(bits) >> 9) & 511
        cnt, last = plsc.scan_count(d)
        dest = plsc.load_gather(offs, [d]) + cnt - 1
        plsc.store_scatter(cb_b, [dest], bits)
        plsc.store_scatter(ci_b, [dest], idxv)
        plsc.addupdate_scatter(offs, [d], cnt, mask=last)
        return c
    lax.fori_loop(0, nvc, rp_body, 0)

    bo16 = plsc.load_gather(hist_p, [jnp.full((16,), t2, jnp.int32)])
    shift16 = jnp.where(half == 0, zero16, bo16)

    def rf_body(i, c):
        bits = cb_b[pl.ds(i * 16, 16)]
        idxv = ci_b[pl.ds(i * 16, 16)]
        d = bits >> 18
        real = d >= t1
        cnt, last = plsc.scan_count(d, mask=real)
        dest = plsc.load_gather(hist_p, [d]) + cnt - 1 - shift16
        safe = jnp.logical_and(real,
                               jnp.logical_and(dest >= 0, dest < CAPT))
        plsc.store_scatter(cb_a, [dest], bits, mask=safe)
        plsc.store_scatter(ci_a, [dest], idxv, mask=safe)
        plsc.addupdate_scatter(hist_p, [d], cnt,
                               mask=jnp.logical_and(last, real))
        return c
    lax.fori_loop(0, nvc, rf_body, 0)

    rdoff = jnp.where(half == 0, 0, KH - jnp.max(bo16))
    lim = jnp.where(half == 0, KH, KL)

    def g_body(i, c):
        for u in range(4):
            k = i * 4 + u
            ridx = rdoff + k * 16 + iota
            idxv = plsc.load_gather(ci_a, [ridx])
            valid = k * 16 + iota < lim
            idxg[pl.ds(k * 16, 16)] = jnp.where(valid, idxv + row * N, zero16)
        return c
    lax.fori_loop(0, KH // 64, g_body, 0)

    cp0 = pltpu.async_copy(lx_hbm.at[idxg], g0, sem)
    cp1 = pltpu.async_copy(ly_hbm.at[idxg], g1, sem)
    cp2 = pltpu.async_copy(sx_hbm.at[idxg], g2, sem)
    cp3 = pltpu.async_copy(sy_hbm.at[idxg], g3, sem)
    cp0.wait(); cp1.wait(); cp2.wait(); cp3.wait()

    def o_body(i, c):
        for u in range(2):
            k = i * 2 + u
            sl = pl.ds(k * 16, 16)
            ridx = rdoff + k * 16 + iota
            idxv = plsc.load_gather(ci_a, [ridx])
            bits = plsc.load_gather(cb_a, [ridx])
            lvl = ((idxv >= OFFS[1]).astype(jnp.int32)
                   + (idxv >= OFFS[2]).astype(jnp.int32)
                   + (idxv >= OFFS[3]).astype(jnp.int32))
            off = jnp.where(lvl == 0, OFFS[0],
                  jnp.where(lvl == 1, OFFS[1],
                  jnp.where(lvl == 2, OFFS[2], OFFS[3])))
            log2w = 7 - lvl
            r = idxv - off
            x = (r & ((1 << log2w) - 1)).astype(jnp.float32)
            y = (r >> log2w).astype(jnp.float32)
            stf = (8 << lvl).astype(jnp.float32)
            g0[sl] = (x + 0.5) * stf + g0[sl] * stf
            g1[sl] = (y + 0.5) * stf + g1[sl] * stf
            g2[sl] = g2[sl] * stf
            g3[sl] = g3[sl] * stf
            gs[sl] = plsc.bitcast(bits, jnp.float32)
        return c
    lax.fori_loop(0, KH // 32, o_body, 0)

    @pl.when(half == 0)
    def _():
        pltpu.sync_copy(g0, ocx.at[row, pl.ds(0, KH)])
        pltpu.sync_copy(g1, ocy.at[row, pl.ds(0, KH)])
        pltpu.sync_copy(g2, ow.at[row, pl.ds(0, KH)])
        pltpu.sync_copy(g3, oh.at[row, pl.ds(0, KH)])
        pltpu.sync_copy(gs, osc.at[row, pl.ds(0, KH)])

    @pl.when(half == 1)
    def _():
        pltpu.sync_copy(g0, ocx.at[row, pl.ds(KH, KH)])
        pltpu.sync_copy(g1, ocy.at[row, pl.ds(KH, KH)])
        pltpu.sync_copy(g2, ow.at[row, pl.ds(KH, KH)])
        pltpu.sync_copy(g3, oh.at[row, pl.ds(KH, KH)])
        pltpu.sync_copy(gs, osc.at[row, pl.ds(KH, KH)])


def kernel(objectness_0, centerness_0, location_0, scale_0,
           objectness_1, centerness_1, location_1, scale_1,
           objectness_2, centerness_2, location_2, scale_2,
           objectness_3, centerness_3, location_3, scale_3):
    objs = [objectness_0, objectness_1, objectness_2, objectness_3]
    ctrs = [centerness_0, centerness_1, centerness_2, centerness_3]
    locs = [location_0, location_1, location_2, location_3]
    scls = [scale_0, scale_1, scale_2, scale_3]

    s_all, lx2, ly2, sx2, sy2 = pl.pallas_call(
        _score_body,
        out_shape=[jax.ShapeDtypeStruct((B, N), jnp.float32)] * 5,
    )(*objs, *ctrs, *locs, *scls)
    lx, ly, sx, sy = (a.reshape(-1) for a in (lx2, ly2, sx2, sy2))

    mesh = plsc.VectorSubcoreMesh(core_axis_name="c", subcore_axis_name="s")
    out_t = [jax.ShapeDtypeStruct((B, KP), jnp.float32)] * 5
    sc_topk = pl.kernel(
        _sc_body,
        out_type=out_t,
        mesh=mesh,
        compiler_params=pltpu.CompilerParams(needs_layout_passes=False),
        scratch_types=[
            pltpu.VMEM((N2,), jnp.float32),
            pltpu.VMEM((H1,), jnp.int32),
            pltpu.VMEM((H1,), jnp.int32),
            pltpu.VMEM((128,), jnp.int32),
            pltpu.VMEM((CAPT,), jnp.int32),
            pltpu.VMEM((CAPT,), jnp.int32),
            pltpu.VMEM((CAPT,), jnp.int32),
            pltpu.VMEM((CAPT,), jnp.int32),
            pltpu.VMEM((512,), jnp.int32),
            pltpu.VMEM((KH,), jnp.int32),
            pltpu.VMEM((KH,), jnp.float32),
            pltpu.VMEM((KH,), jnp.float32),
            pltpu.VMEM((KH,), jnp.float32),
            pltpu.VMEM((KH,), jnp.float32),
            pltpu.VMEM((KH,), jnp.float32),
            pltpu.VMEM_SHARED((16, H1), jnp.int32),
            pltpu.VMEM_SHARED((16, CAPH), jnp.int32),
            pltpu.VMEM_SHARED((16, CAPH), jnp.int32),
            pltpu.VMEM_SHARED((16, 128), jnp.int32),
            pltpu.SemaphoreType.DMA,
        ],
    )
    cx, cy, w, h, s = sc_topk(s_all.reshape(B * 2, N2), lx, ly, sx, sy)
    return jnp.stack([cx, cy, w, h, s], axis=-1)[:, :K, :]
